# baseline (speedup 1.0000x reference)
import functools

import jax
import jax.numpy as jnp
from jax import lax
from jax.experimental import pallas as pl
from jax.experimental.pallas import tpu as pltpu
from jax.experimental.pallas import tpu_sc as plsc

B = 1024
L = 20
D = 128
LAYER_SIZES = [512, 256, 128]
SCALAR_FEATS = ['pl_collaborative', 'pl_pid', 'duration_ms_seed', 'n_songs',
                'n_artists', 'n_albums']
SEQ_FEATS = ['artist_name', 'track_uri', 'track_name', 'duration_ms_songs',
             'album_name', 'artist_pop', 'artists_followers', 'track_pop',
             'artist_genres']

NC = 2
NS = 16
NW = NC * NS
NBIG = 10
ROWS_BIG = L * B
KCH = ROWS_BIG // NW // 128
NSC = 6


RPW = ROWS_BIG // NW
SCH = 256
NSW = NSC * (B // SCH)


def _sc_gather_body(*refs):
    tabs = refs[0:NBIG]
    idx_all = refs[NBIG]
    stabs = refs[NBIG + 1:NBIG + 1 + NSC]
    sidx = refs[NBIG + 1 + NSC]
    outs = refs[NBIG + 2 + NSC:2 * NBIG + 2 + NSC]
    souts = refs[2 * NBIG + 2 + NSC:2 * NBIG + 2 + 2 * NSC]
    rest = refs[2 * NBIG + 2 + 2 * NSC:]
    idx_fs = rest[0:NBIG]
    rows_v, idx_s, gsem, ssem, s2sem = rest[NBIG:]

    w = lax.axis_index("s") * NC + lax.axis_index("c")

    for f in range(NBIG):
        pltpu.sync_copy(idx_all.at[w, f], idx_fs[f])

    for f in range(NBIG):
        g = pltpu.make_async_copy(tabs[f].at[idx_fs[f]], rows_v, gsem)
        g.start()
        g.wait()
        s = pltpu.make_async_copy(rows_v, outs[f].at[w], ssem)
        s.start()
        s.wait()

    rows_sv = rows_v.at[pl.ds(0, SCH)]
    for f in range(NSC):
        for q in range(B // SCH):
            @pl.when(w == f * (B // SCH) + q)
            def _(f=f, q=q):
                pltpu.sync_copy(sidx.at[f * (B // SCH) + q], idx_s)
                g = pltpu.make_async_copy(stabs[f].at[idx_s], rows_sv, s2sem)
                g.start()
                g.wait()
                pltpu.sync_copy(rows_sv, souts[f].at[q])


def _sc_gather(tabs, idx_all, stabs, sidx):
    out_type = ([jax.ShapeDtypeStruct((NW, RPW, D), jnp.float32)
                 for _ in range(NBIG)]
                + [jax.ShapeDtypeStruct((B // SCH, SCH, D), jnp.float32)
                   for _ in range(NSC)])
    mesh = plsc.VectorSubcoreMesh(core_axis_name="c", subcore_axis_name="s")
    fn = pl.kernel(
        _sc_gather_body,
        out_type=out_type,
        mesh=mesh,
        scratch_types=(
            [pltpu.VMEM((RPW,), jnp.int32) for _ in range(NBIG)]
            + [
                pltpu.VMEM((RPW, D), jnp.float32),
                pltpu.VMEM((SCH,), jnp.int32),
                pltpu.SemaphoreType.DMA,
                pltpu.SemaphoreType.DMA,
                pltpu.SemaphoreType.DMA,
            ]
        ),
    )
    return fn(*tabs, idx_all, *stabs, sidx)


def _tc_body(BB, *refs):
    name_ref = refs[0]
    scal = refs[1:1 + NSC]
    seqs = refs[1 + NSC:1 + NSC + 9]
    wxs = refs[1 + NSC + 9:1 + NSC + 18]
    whs = refs[1 + NSC + 18:1 + NSC + 27]
    bs = refs[1 + NSC + 27:1 + NSC + 36]
    W0, b0, W1, b1, W2, b2 = refs[1 + NSC + 36:1 + NSC + 42]
    out_ref = refs[1 + NSC + 42]
    (x_ref,) = refs[1 + NSC + 43:]

    f32 = jnp.float32
    x_ref[:, 0:D] = jnp.mean(name_ref[...], axis=0)
    for j in range(NSC):
        x_ref[:, (1 + j) * D:(2 + j) * D] = scal[j][...]

    def step(t, hs):
        new = []
        for f in range(9):
            h = hs[f]
            xt = (jnp.dot(seqs[f][t], wxs[f][...], preferred_element_type=f32)
                  + bs[f][...])
            hg = jnp.dot(h, whs[f][:, :2 * D], preferred_element_type=f32)
            z = jax.nn.sigmoid(xt[:, :D] + hg[:, :D])
            r = jax.nn.sigmoid(xt[:, D:2 * D] + hg[:, D:])
            hh = jnp.tanh(xt[:, 2 * D:]
                          + jnp.dot(r * h, whs[f][:, 2 * D:],
                                    preferred_element_type=f32))
            new.append(z * h + (1.0 - z) * hh)
        return tuple(new)

    hs = lax.fori_loop(0, L, step,
                       tuple(jnp.zeros((BB, D), f32) for _ in range(9)))
    for f in range(9):
        x_ref[:, (7 + f) * D:(8 + f) * D] = hs[f]

    x = x_ref[...]
    y = jax.nn.relu(jnp.dot(x, W0[...], preferred_element_type=f32) + b0[...])
    y = jax.nn.relu(jnp.dot(y, W1[...], preferred_element_type=f32) + b1[...])
    out_ref[...] = jnp.dot(y, W2[...], preferred_element_type=f32) + b2[...]


def _tc_forward(name_g, scal_g, seq_g, wxs, whs, bs, dense):
    BB = 128
    grid = (B // BB,)
    time_spec = pl.BlockSpec((L, BB, D), lambda i: (0, i, 0))
    row_spec = pl.BlockSpec((BB, D), lambda i: (i, 0))

    def full(shape):
        n = len(shape)
        return pl.BlockSpec(shape, lambda i, n=n: (0,) * n)

    in_specs = ([time_spec] + [row_spec] * NSC + [time_spec] * 9
                + [full((D, 3 * D))] * 9 + [full((D, 3 * D))] * 9
                + [full((3 * D,))] * 9
                + [full(d.shape) for d in dense])
    out_spec = pl.BlockSpec((BB, LAYER_SIZES[-1]), lambda i: (i, 0))

    return pl.pallas_call(
        functools.partial(_tc_body, BB),
        grid=grid,
        in_specs=in_specs,
        out_specs=out_spec,
        out_shape=jax.ShapeDtypeStruct((B, LAYER_SIZES[-1]), jnp.float32),
        scratch_shapes=[
            pltpu.VMEM((BB, 16 * D), jnp.float32),
        ],
    )(name_g, *scal_g, *seq_g, *wxs, *whs, *bs, *dense)


def kernel(pl_name_tokens, pl_collaborative_idx, pl_pid_idx,
           duration_ms_seed_idx, n_songs_idx, n_artists_idx, n_albums_idx,
           artist_name_seq, track_uri_seq, track_name_seq,
           duration_ms_songs_seq, album_name_seq, artist_pop_seq,
           artists_followers_seq, track_pop_seq, artist_genres_seq, params):
    seq_idx = [artist_name_seq, track_uri_seq, track_name_seq,
               duration_ms_songs_seq, album_name_seq, artist_pop_seq,
               artists_followers_seq, track_pop_seq, artist_genres_seq]
    scal_idx = [pl_collaborative_idx, pl_pid_idx, duration_ms_seed_idx,
                n_songs_idx, n_artists_idx, n_albums_idx]

    big_names = ['pl_name'] + SEQ_FEATS
    big_idx = [pl_name_tokens] + seq_idx
    idxs = [jnp.reshape(jnp.swapaxes(a, 0, 1).astype(jnp.int32), (NW, RPW))
            for a in big_idx]
    idx_all = jnp.stack(idxs, axis=1)
    sidx = jnp.concatenate([jnp.reshape(a.astype(jnp.int32), (B // SCH, SCH))
                            for a in scal_idx], axis=0)
    tabs = [params['tab_' + n] for n in big_names]
    stabs = [params['tab_' + n] for n in SCALAR_FEATS]

    g = _sc_gather(tabs, idx_all, stabs, sidx)
    big_g = [jnp.reshape(a, (L, B, D)) for a in g[:NBIG]]
    scal_g = [jnp.reshape(a, (B, D)) for a in g[NBIG:]]

    wxs = [params[f + '_Wx'] for f in SEQ_FEATS]
    whs = [params[f + '_Wh'] for f in SEQ_FEATS]
    bs = [params[f + '_b'] for f in SEQ_FEATS]
    dense = [params['dense_W0'], params['dense_b0'],
             params['dense_W1'], params['dense_b1'],
             params['dense_W2'], params['dense_b2']]

    return _tc_forward(big_g[0], scal_g, big_g[1:], wxs, whs, bs, dense)


# speedup vs baseline: 1.0006x; 1.0006x over previous
import functools

import jax
import jax.numpy as jnp
from jax import lax
from jax.experimental import pallas as pl
from jax.experimental.pallas import tpu as pltpu
from jax.experimental.pallas import tpu_sc as plsc

B = 1024
L = 20
D = 128
LAYER_SIZES = [512, 256, 128]
SCALAR_FEATS = ['pl_collaborative', 'pl_pid', 'duration_ms_seed', 'n_songs',
                'n_artists', 'n_albums']
SEQ_FEATS = ['artist_name', 'track_uri', 'track_name', 'duration_ms_songs',
             'album_name', 'artist_pop', 'artists_followers', 'track_pop',
             'artist_genres']

NC = 2
NS = 16
NW = NC * NS
NBIG = 10
ROWS_BIG = L * B
KCH = ROWS_BIG // NW // 128
NSC = 6


RPW = ROWS_BIG // NW
SCH = 256
NSW = NSC * (B // SCH)


def _sc_gather_body(*refs):
    tabs = refs[0:NBIG]
    idx_all = refs[NBIG]
    stabs = refs[NBIG + 1:NBIG + 1 + NSC]
    sidx = refs[NBIG + 1 + NSC]
    outs = refs[NBIG + 2 + NSC:2 * NBIG + 2 + NSC]
    souts = refs[2 * NBIG + 2 + NSC:2 * NBIG + 2 + 2 * NSC]
    rest = refs[2 * NBIG + 2 + 2 * NSC:]
    idx_fs = rest[0:NBIG]
    rows_v, idx_s, gsem, ssem, s2sem = rest[NBIG:]

    w = lax.axis_index("s") * NC + lax.axis_index("c")

    for f in range(NBIG):
        pltpu.sync_copy(idx_all.at[w, f], idx_fs[f])

    for f in range(NBIG):
        g = pltpu.make_async_copy(tabs[f].at[idx_fs[f]], rows_v, gsem)
        g.start()
        g.wait()
        s = pltpu.make_async_copy(rows_v, outs[f].at[w], ssem)
        s.start()
        s.wait()

    rows_sv = rows_v.at[pl.ds(0, SCH)]
    for f in range(NSC):
        for q in range(B // SCH):
            @pl.when(w == f * (B // SCH) + q)
            def _(f=f, q=q):
                pltpu.sync_copy(sidx.at[f * (B // SCH) + q], idx_s)
                g = pltpu.make_async_copy(stabs[f].at[idx_s], rows_sv, s2sem)
                g.start()
                g.wait()
                pltpu.sync_copy(rows_sv, souts[f].at[q])


def _sc_gather(tabs, idx_all, stabs, sidx):
    out_type = ([jax.ShapeDtypeStruct((NW, RPW, D), jnp.float32)
                 for _ in range(NBIG)]
                + [jax.ShapeDtypeStruct((B // SCH, SCH, D), jnp.float32)
                   for _ in range(NSC)])
    mesh = plsc.VectorSubcoreMesh(core_axis_name="c", subcore_axis_name="s")
    fn = pl.kernel(
        _sc_gather_body,
        out_type=out_type,
        mesh=mesh,
        scratch_types=(
            [pltpu.VMEM((RPW,), jnp.int32) for _ in range(NBIG)]
            + [
                pltpu.VMEM((RPW, D), jnp.float32),
                pltpu.VMEM((SCH,), jnp.int32),
                pltpu.SemaphoreType.DMA,
                pltpu.SemaphoreType.DMA,
                pltpu.SemaphoreType.DMA,
            ]
        ),
    )
    return fn(*tabs, idx_all, *stabs, sidx)


def _tc_body(BB, *refs):
    name_ref = refs[0]
    scal = refs[1:1 + NSC]
    seqs = refs[1 + NSC:1 + NSC + 9]
    wxs = refs[1 + NSC + 9:1 + NSC + 18]
    whs = refs[1 + NSC + 18:1 + NSC + 27]
    bs = refs[1 + NSC + 27:1 + NSC + 36]
    W0, b0, W1, b1, W2, b2 = refs[1 + NSC + 36:1 + NSC + 42]
    out_ref = refs[1 + NSC + 42]
    (x_ref,) = refs[1 + NSC + 43:]

    f32 = jnp.float32
    x_ref[:, 0:D] = jnp.mean(name_ref[...], axis=0)
    for j in range(NSC):
        x_ref[:, (1 + j) * D:(2 + j) * D] = scal[j][...]

    bf16 = jnp.bfloat16
    wxs_b = [wxs[f][...].astype(bf16) for f in range(9)]
    whg_b = [whs[f][:, :2 * D].astype(bf16) for f in range(9)]
    whh_b = [whs[f][:, 2 * D:].astype(bf16) for f in range(9)]

    def step(t, hs):
        new = []
        for f in range(9):
            h = hs[f]
            hb = h.astype(bf16)
            xt = (jnp.dot(seqs[f][t].astype(bf16), wxs_b[f],
                          preferred_element_type=f32) + bs[f][...])
            hg = jnp.dot(hb, whg_b[f], preferred_element_type=f32)
            z = jax.nn.sigmoid(xt[:, :D] + hg[:, :D])
            r = jax.nn.sigmoid(xt[:, D:2 * D] + hg[:, D:])
            hh = jnp.tanh(xt[:, 2 * D:]
                          + jnp.dot((r * h).astype(bf16), whh_b[f],
                                    preferred_element_type=f32))
            new.append(z * h + (1.0 - z) * hh)
        return tuple(new)

    hs = lax.fori_loop(0, L, step,
                       tuple(jnp.zeros((BB, D), f32) for _ in range(9)))
    for f in range(9):
        x_ref[:, (7 + f) * D:(8 + f) * D] = hs[f]

    x = x_ref[...].astype(bf16)
    y = jax.nn.relu(jnp.dot(x, W0[...].astype(bf16),
                            preferred_element_type=f32) + b0[...])
    y = jax.nn.relu(jnp.dot(y.astype(bf16), W1[...].astype(bf16),
                            preferred_element_type=f32) + b1[...])
    out_ref[...] = jnp.dot(y.astype(bf16), W2[...].astype(bf16),
                           preferred_element_type=f32) + b2[...]


def _tc_forward(name_g, scal_g, seq_g, wxs, whs, bs, dense):
    BB = 128
    grid = (B // BB,)
    time_spec = pl.BlockSpec((L, BB, D), lambda i: (0, i, 0))
    row_spec = pl.BlockSpec((BB, D), lambda i: (i, 0))

    def full(shape):
        n = len(shape)
        return pl.BlockSpec(shape, lambda i, n=n: (0,) * n)

    in_specs = ([time_spec] + [row_spec] * NSC + [time_spec] * 9
                + [full((D, 3 * D))] * 9 + [full((D, 3 * D))] * 9
                + [full((3 * D,))] * 9
                + [full(d.shape) for d in dense])
    out_spec = pl.BlockSpec((BB, LAYER_SIZES[-1]), lambda i: (i, 0))

    return pl.pallas_call(
        functools.partial(_tc_body, BB),
        grid=grid,
        in_specs=in_specs,
        out_specs=out_spec,
        out_shape=jax.ShapeDtypeStruct((B, LAYER_SIZES[-1]), jnp.float32),
        scratch_shapes=[
            pltpu.VMEM((BB, 16 * D), jnp.float32),
        ],
    )(name_g, *scal_g, *seq_g, *wxs, *whs, *bs, *dense)


def kernel(pl_name_tokens, pl_collaborative_idx, pl_pid_idx,
           duration_ms_seed_idx, n_songs_idx, n_artists_idx, n_albums_idx,
           artist_name_seq, track_uri_seq, track_name_seq,
           duration_ms_songs_seq, album_name_seq, artist_pop_seq,
           artists_followers_seq, track_pop_seq, artist_genres_seq, params):
    seq_idx = [artist_name_seq, track_uri_seq, track_name_seq,
               duration_ms_songs_seq, album_name_seq, artist_pop_seq,
               artists_followers_seq, track_pop_seq, artist_genres_seq]
    scal_idx = [pl_collaborative_idx, pl_pid_idx, duration_ms_seed_idx,
                n_songs_idx, n_artists_idx, n_albums_idx]

    big_names = ['pl_name'] + SEQ_FEATS
    big_idx = [pl_name_tokens] + seq_idx
    idxs = [jnp.reshape(jnp.swapaxes(a, 0, 1).astype(jnp.int32), (NW, RPW))
            for a in big_idx]
    idx_all = jnp.stack(idxs, axis=1)
    sidx = jnp.concatenate([jnp.reshape(a.astype(jnp.int32), (B // SCH, SCH))
                            for a in scal_idx], axis=0)
    tabs = [params['tab_' + n] for n in big_names]
    stabs = [params['tab_' + n] for n in SCALAR_FEATS]

    g = _sc_gather(tabs, idx_all, stabs, sidx)
    big_g = [jnp.reshape(a, (L, B, D)) for a in g[:NBIG]]
    scal_g = [jnp.reshape(a, (B, D)) for a in g[NBIG:]]

    wxs = [params[f + '_Wx'] for f in SEQ_FEATS]
    whs = [params[f + '_Wh'] for f in SEQ_FEATS]
    bs = [params[f + '_b'] for f in SEQ_FEATS]
    dense = [params['dense_W0'], params['dense_b0'],
             params['dense_W1'], params['dense_b1'],
             params['dense_W2'], params['dense_b2']]

    return _tc_forward(big_g[0], scal_g, big_g[1:], wxs, whs, bs, dense)


# speedup vs baseline: 1.0314x; 1.0308x over previous
import functools

import jax
import jax.numpy as jnp
from jax import lax
from jax.experimental import pallas as pl
from jax.experimental.pallas import tpu as pltpu
from jax.experimental.pallas import tpu_sc as plsc

B = 1024
L = 20
D = 128
LAYER_SIZES = [512, 256, 128]
SCALAR_FEATS = ['pl_collaborative', 'pl_pid', 'duration_ms_seed', 'n_songs',
                'n_artists', 'n_albums']
SEQ_FEATS = ['artist_name', 'track_uri', 'track_name', 'duration_ms_songs',
             'album_name', 'artist_pop', 'artists_followers', 'track_pop',
             'artist_genres']

NC = 2
NS = 16
NW = NC * NS
NBIG = 10
ROWS_BIG = L * B
KCH = ROWS_BIG // NW // 128
NSC = 6


RPW = ROWS_BIG // NW
SCH = 256
NSW = NSC * (B // SCH)


def _sc_gather_body(*refs):
    tabs = refs[0:NBIG]
    idx_all = refs[NBIG]
    stabs = refs[NBIG + 1:NBIG + 1 + NSC]
    sidx = refs[NBIG + 1 + NSC]
    outs = refs[NBIG + 2 + NSC:2 * NBIG + 2 + NSC]
    souts = refs[2 * NBIG + 2 + NSC:2 * NBIG + 2 + 2 * NSC]
    rest = refs[2 * NBIG + 2 + 2 * NSC:]
    idx_fs = rest[0:NBIG]
    rows_v, idx_s, gsem, ssem, s2sem = rest[NBIG:]

    w = lax.axis_index("s") * NC + lax.axis_index("c")

    for f in range(NBIG):
        pltpu.sync_copy(idx_all.at[w, f], idx_fs[f])

    for f in range(NBIG):
        g = pltpu.make_async_copy(tabs[f].at[idx_fs[f]], rows_v, gsem)
        g.start()
        g.wait()
        s = pltpu.make_async_copy(rows_v, outs[f].at[w], ssem)
        s.start()
        s.wait()

    rows_sv = rows_v.at[pl.ds(0, SCH)]
    for f in range(NSC):
        for q in range(B // SCH):
            @pl.when(w == f * (B // SCH) + q)
            def _(f=f, q=q):
                pltpu.sync_copy(sidx.at[f * (B // SCH) + q], idx_s)
                g = pltpu.make_async_copy(stabs[f].at[idx_s], rows_sv, s2sem)
                g.start()
                g.wait()
                pltpu.sync_copy(rows_sv, souts[f].at[q])


def _sc_gather(tabs, idx_all, stabs, sidx):
    out_type = ([jax.ShapeDtypeStruct((NW, RPW, D), jnp.float32)
                 for _ in range(NBIG)]
                + [jax.ShapeDtypeStruct((B // SCH, SCH, D), jnp.float32)
                   for _ in range(NSC)])
    mesh = plsc.VectorSubcoreMesh(core_axis_name="c", subcore_axis_name="s")
    fn = pl.kernel(
        _sc_gather_body,
        out_type=out_type,
        mesh=mesh,
        scratch_types=(
            [pltpu.VMEM((RPW,), jnp.int32) for _ in range(NBIG)]
            + [
                pltpu.VMEM((RPW, D), jnp.float32),
                pltpu.VMEM((SCH,), jnp.int32),
                pltpu.SemaphoreType.DMA,
                pltpu.SemaphoreType.DMA,
                pltpu.SemaphoreType.DMA,
            ]
        ),
    )
    return fn(*tabs, idx_all, *stabs, sidx)


def _tc_body(BB, *refs):
    name_ref = refs[0]
    scal = refs[1:1 + NSC]
    seqs = refs[1 + NSC:1 + NSC + 9]
    wxs = refs[1 + NSC + 9:1 + NSC + 18]
    whs = refs[1 + NSC + 18:1 + NSC + 27]
    bs = refs[1 + NSC + 27:1 + NSC + 36]
    W0, b0, W1, b1, W2, b2 = refs[1 + NSC + 36:1 + NSC + 42]
    out_ref = refs[1 + NSC + 42]
    x_ref, xz_ref = refs[1 + NSC + 43:]

    f32 = jnp.float32
    x_ref[:, 0:D] = jnp.mean(name_ref[...], axis=0)
    for j in range(NSC):
        x_ref[:, (1 + j) * D:(2 + j) * D] = scal[j][...]

    bf16 = jnp.bfloat16
    whg_b = [whs[f][:, :2 * D].astype(bf16) for f in range(9)]
    whh_b = [whs[f][:, 2 * D:].astype(bf16) for f in range(9)]

    for f in range(9):
        seq = jnp.reshape(seqs[f][...], (L * BB, D)).astype(bf16)
        xz = (jnp.dot(seq, wxs[f][...].astype(bf16),
                      preferred_element_type=f32) + bs[f][...])
        xz_ref[f] = jnp.reshape(xz, (L, BB, 3 * D)).astype(bf16)

    def step(t, hs):
        new = []
        for f in range(9):
            h = hs[f]
            xt = xz_ref[f, t].astype(f32)
            hg = jnp.dot(h.astype(bf16), whg_b[f], preferred_element_type=f32)
            z = jax.nn.sigmoid(xt[:, :D] + hg[:, :D])
            r = jax.nn.sigmoid(xt[:, D:2 * D] + hg[:, D:])
            hh = jnp.tanh(xt[:, 2 * D:]
                          + jnp.dot((r * h).astype(bf16), whh_b[f],
                                    preferred_element_type=f32))
            new.append(z * h + (1.0 - z) * hh)
        return tuple(new)

    hs = lax.fori_loop(0, L, step,
                       tuple(jnp.zeros((BB, D), f32) for _ in range(9)))
    for f in range(9):
        x_ref[:, (7 + f) * D:(8 + f) * D] = hs[f]

    x = x_ref[...].astype(bf16)
    y = jax.nn.relu(jnp.dot(x, W0[...].astype(bf16),
                            preferred_element_type=f32) + b0[...])
    y = jax.nn.relu(jnp.dot(y.astype(bf16), W1[...].astype(bf16),
                            preferred_element_type=f32) + b1[...])
    out_ref[...] = jnp.dot(y.astype(bf16), W2[...].astype(bf16),
                           preferred_element_type=f32) + b2[...]


def _tc_forward(name_g, scal_g, seq_g, wxs, whs, bs, dense):
    BB = 128
    grid = (B // BB,)
    time_spec = pl.BlockSpec((L, BB, D), lambda i: (0, i, 0))
    row_spec = pl.BlockSpec((BB, D), lambda i: (i, 0))

    def full(shape):
        n = len(shape)
        return pl.BlockSpec(shape, lambda i, n=n: (0,) * n)

    in_specs = ([time_spec] + [row_spec] * NSC + [time_spec] * 9
                + [full((D, 3 * D))] * 9 + [full((D, 3 * D))] * 9
                + [full((3 * D,))] * 9
                + [full(d.shape) for d in dense])
    out_spec = pl.BlockSpec((BB, LAYER_SIZES[-1]), lambda i: (i, 0))

    return pl.pallas_call(
        functools.partial(_tc_body, BB),
        grid=grid,
        in_specs=in_specs,
        out_specs=out_spec,
        out_shape=jax.ShapeDtypeStruct((B, LAYER_SIZES[-1]), jnp.float32),
        scratch_shapes=[
            pltpu.VMEM((BB, 16 * D), jnp.float32),
            pltpu.VMEM((9, L, BB, 3 * D), jnp.bfloat16),
        ],
    )(name_g, *scal_g, *seq_g, *wxs, *whs, *bs, *dense)


def kernel(pl_name_tokens, pl_collaborative_idx, pl_pid_idx,
           duration_ms_seed_idx, n_songs_idx, n_artists_idx, n_albums_idx,
           artist_name_seq, track_uri_seq, track_name_seq,
           duration_ms_songs_seq, album_name_seq, artist_pop_seq,
           artists_followers_seq, track_pop_seq, artist_genres_seq, params):
    seq_idx = [artist_name_seq, track_uri_seq, track_name_seq,
               duration_ms_songs_seq, album_name_seq, artist_pop_seq,
               artists_followers_seq, track_pop_seq, artist_genres_seq]
    scal_idx = [pl_collaborative_idx, pl_pid_idx, duration_ms_seed_idx,
                n_songs_idx, n_artists_idx, n_albums_idx]

    big_names = ['pl_name'] + SEQ_FEATS
    big_idx = [pl_name_tokens] + seq_idx
    idxs = [jnp.reshape(jnp.swapaxes(a, 0, 1).astype(jnp.int32), (NW, RPW))
            for a in big_idx]
    idx_all = jnp.stack(idxs, axis=1)
    sidx = jnp.concatenate([jnp.reshape(a.astype(jnp.int32), (B // SCH, SCH))
                            for a in scal_idx], axis=0)
    tabs = [params['tab_' + n] for n in big_names]
    stabs = [params['tab_' + n] for n in SCALAR_FEATS]

    g = _sc_gather(tabs, idx_all, stabs, sidx)
    big_g = [jnp.reshape(a, (L, B, D)) for a in g[:NBIG]]
    scal_g = [jnp.reshape(a, (B, D)) for a in g[NBIG:]]

    wxs = [params[f + '_Wx'] for f in SEQ_FEATS]
    whs = [params[f + '_Wh'] for f in SEQ_FEATS]
    bs = [params[f + '_b'] for f in SEQ_FEATS]
    dense = [params['dense_W0'], params['dense_b0'],
             params['dense_W1'], params['dense_b1'],
             params['dense_W2'], params['dense_b2']]

    return _tc_forward(big_g[0], scal_g, big_g[1:], wxs, whs, bs, dense)


# speedup vs baseline: 1.6847x; 1.6334x over previous
import functools

import jax
import jax.numpy as jnp
from jax import lax
from jax.experimental import pallas as pl
from jax.experimental.pallas import tpu as pltpu
from jax.experimental.pallas import tpu_sc as plsc

B = 1024
L = 20
D = 128
LAYER_SIZES = [512, 256, 128]
SCALAR_FEATS = ['pl_collaborative', 'pl_pid', 'duration_ms_seed', 'n_songs',
                'n_artists', 'n_albums']
SEQ_FEATS = ['artist_name', 'track_uri', 'track_name', 'duration_ms_songs',
             'album_name', 'artist_pop', 'artists_followers', 'track_pop',
             'artist_genres']
GATH_SEQ = ['artist_name', 'track_uri', 'track_name', 'album_name',
            'artist_genres']
ONEHOT_SEQ = ['duration_ms_songs', 'artist_pop', 'artists_followers',
              'track_pop']
ONEHOT_V = {'duration_ms_songs': 101, 'artist_pop': 11,
            'artists_followers': 11, 'track_pop': 11}
SLOT = {'pl_name': 0, 'pl_collaborative': 1, 'pl_pid': 2,
        'duration_ms_seed': 3, 'n_songs': 4, 'n_artists': 5, 'n_albums': 6,
        'artist_name': 7, 'track_uri': 8, 'track_name': 9,
        'duration_ms_songs': 10, 'album_name': 11, 'artist_pop': 12,
        'artists_followers': 13, 'track_pop': 14, 'artist_genres': 15}

NC = 2
NS = 16
NW = NC * NS
NBIG = 1 + len(GATH_SEQ)
ROWS_BIG = L * B
RPW = ROWS_BIG // NW
SCH = 256
NSC = 6
NSW = NSC * (B // SCH)
NG = len(GATH_SEQ)
NO = len(ONEHOT_SEQ)


def _sc_gather_body(*refs):
    tabs = refs[0:NBIG]
    idx_all = refs[NBIG]
    stabs = refs[NBIG + 1:NBIG + 1 + NSC]
    sidx = refs[NBIG + 1 + NSC]
    outs = refs[NBIG + 2 + NSC:2 * NBIG + 2 + NSC]
    souts = refs[2 * NBIG + 2 + NSC:2 * NBIG + 2 + 2 * NSC]
    rest = refs[2 * NBIG + 2 + 2 * NSC:]
    idx_fs = rest[0:NBIG]
    rows_v, idx_s, gsem, ssem, s2sem = rest[NBIG:]

    w = lax.axis_index("s") * NC + lax.axis_index("c")

    for f in range(NBIG):
        pltpu.sync_copy(idx_all.at[w, f], idx_fs[f])

    for f in range(NBIG):
        g = pltpu.make_async_copy(tabs[f].at[idx_fs[f]], rows_v, gsem)
        g.start()
        g.wait()
        s = pltpu.make_async_copy(rows_v, outs[f].at[w], ssem)
        s.start()
        s.wait()

    rows_sv = rows_v.at[pl.ds(0, SCH)]
    for f in range(NSC):
        for q in range(B // SCH):
            @pl.when(w == f * (B // SCH) + q)
            def _(f=f, q=q):
                pltpu.sync_copy(sidx.at[f * (B // SCH) + q], idx_s)
                g = pltpu.make_async_copy(stabs[f].at[idx_s], rows_sv, s2sem)
                g.start()
                g.wait()
                pltpu.sync_copy(rows_sv, souts[f].at[q])


def _sc_gather(tabs, idx_all, stabs, sidx):
    out_type = ([jax.ShapeDtypeStruct((NW, RPW, D), jnp.float32)
                 for _ in range(NBIG)]
                + [jax.ShapeDtypeStruct((B // SCH, SCH, D), jnp.float32)
                   for _ in range(NSC)])
    mesh = plsc.VectorSubcoreMesh(core_axis_name="c", subcore_axis_name="s")
    fn = pl.kernel(
        _sc_gather_body,
        out_type=out_type,
        mesh=mesh,
        scratch_types=(
            [pltpu.VMEM((RPW,), jnp.int32) for _ in range(NBIG)]
            + [
                pltpu.VMEM((RPW, D), jnp.float32),
                pltpu.VMEM((SCH,), jnp.int32),
                pltpu.SemaphoreType.DMA,
                pltpu.SemaphoreType.DMA,
                pltpu.SemaphoreType.DMA,
            ]
        ),
    )
    return fn(*tabs, idx_all, *stabs, sidx)


def _gru_loop(xz_ref, whg_b, whh_b, n, BB):
    f32 = jnp.float32
    bf16 = jnp.bfloat16

    def step(t, hs):
        new = []
        for f in range(n):
            h = hs[f]
            xt = xz_ref[f, t].astype(f32)
            hg = jnp.dot(h.astype(bf16), whg_b[f], preferred_element_type=f32)
            z = jax.nn.sigmoid(xt[:, :D] + hg[:, :D])
            r = jax.nn.sigmoid(xt[:, D:2 * D] + hg[:, D:])
            hh = jnp.tanh(xt[:, 2 * D:]
                          + jnp.dot((r * h).astype(bf16), whh_b[f],
                                    preferred_element_type=f32))
            new.append(z * h + (1.0 - z) * hh)
        return tuple(new)

    return lax.fori_loop(0, L, step,
                         tuple(jnp.zeros((BB, D), f32) for _ in range(n)))


def _tc1_body(BB, *refs):
    idxs = refs[0:NO]
    tabs = refs[NO:2 * NO]
    wxs = refs[2 * NO:3 * NO]
    whs = refs[3 * NO:4 * NO]
    bs = refs[4 * NO:5 * NO]
    out_ref = refs[5 * NO]
    xz_ref = refs[5 * NO + 1]

    f32 = jnp.float32
    bf16 = jnp.bfloat16
    whg_b = [whs[f][:, :2 * D].astype(bf16) for f in range(NO)]
    whh_b = [whs[f][:, 2 * D:].astype(bf16) for f in range(NO)]

    for f, name in enumerate(ONEHOT_SEQ):
        V = ONEHOT_V[name]
        tabwx = (jnp.dot(tabs[f][...], wxs[f][...],
                         preferred_element_type=f32)).astype(bf16)
        for l in range(L):
            iv = idxs[f][l]
            oh = (lax.broadcasted_iota(jnp.int32, (BB, V), 1)
                  == iv).astype(bf16)
            xz = jnp.dot(oh, tabwx, preferred_element_type=f32) + bs[f][...]
            xz_ref[f, l] = xz.astype(bf16)

    hs = _gru_loop(xz_ref, whg_b, whh_b, NO, BB)
    for f in range(NO):
        out_ref[:, f * D:(f + 1) * D] = hs[f]


def _tc1(idxs, tabs, wxs, whs, bs):
    BB = 128
    grid = (B // BB,)

    def full(shape):
        n = len(shape)
        return pl.BlockSpec(shape, lambda i, n=n: (0,) * n)

    in_specs = ([pl.BlockSpec((L, BB, 1), lambda i: (0, i, 0))
                 for _ in range(NO)]
                + [full(t.shape) for t in tabs]
                + [full((D, 3 * D))] * NO
                + [full((D, 3 * D))] * NO
                + [full((3 * D,))] * NO)
    return pl.pallas_call(
        functools.partial(_tc1_body, BB),
        grid=grid,
        in_specs=in_specs,
        out_specs=pl.BlockSpec((BB, NO * D), lambda i: (i, 0)),
        out_shape=jax.ShapeDtypeStruct((B, NO * D), jnp.float32),
        scratch_shapes=[
            pltpu.VMEM((NO, L, BB, 3 * D), jnp.bfloat16),
        ],
    )(*idxs, *tabs, *wxs, *whs, *bs)


def _tc2_body(BB, *refs):
    name_ref = refs[0]
    scal = refs[1:1 + NSC]
    seqs = refs[1 + NSC:1 + NSC + NG]
    h4_ref = refs[1 + NSC + NG]
    wxs = refs[2 + NSC + NG:2 + NSC + 2 * NG]
    whs = refs[2 + NSC + 2 * NG:2 + NSC + 3 * NG]
    bs = refs[2 + NSC + 3 * NG:2 + NSC + 4 * NG]
    W0, b0, W1, b1, W2, b2 = refs[2 + NSC + 4 * NG:8 + NSC + 4 * NG]
    out_ref = refs[8 + NSC + 4 * NG]
    x_ref, xz_ref = refs[9 + NSC + 4 * NG:]

    f32 = jnp.float32
    bf16 = jnp.bfloat16

    x_ref[:, 0:D] = jnp.mean(name_ref[...], axis=0)
    for j in range(NSC):
        x_ref[:, (1 + j) * D:(2 + j) * D] = scal[j][...]
    h4 = h4_ref[...]
    x_ref[:, 10 * D:11 * D] = h4[:, 0:D]
    x_ref[:, 12 * D:15 * D] = h4[:, D:4 * D]

    whg_b = [whs[f][:, :2 * D].astype(bf16) for f in range(NG)]
    whh_b = [whs[f][:, 2 * D:].astype(bf16) for f in range(NG)]

    for f in range(NG):
        seq = jnp.reshape(seqs[f][...], (L * BB, D)).astype(bf16)
        xz = (jnp.dot(seq, wxs[f][...].astype(bf16),
                      preferred_element_type=f32) + bs[f][...])
        xz_ref[f] = jnp.reshape(xz, (L, BB, 3 * D)).astype(bf16)

    hs = _gru_loop(xz_ref, whg_b, whh_b, NG, BB)
    for f, name in enumerate(GATH_SEQ):
        s = SLOT[name]
        x_ref[:, s * D:(s + 1) * D] = hs[f]

    x = x_ref[...].astype(bf16)
    y = jax.nn.relu(jnp.dot(x, W0[...].astype(bf16),
                            preferred_element_type=f32) + b0[...])
    y = jax.nn.relu(jnp.dot(y.astype(bf16), W1[...].astype(bf16),
                            preferred_element_type=f32) + b1[...])
    out_ref[...] = jnp.dot(y.astype(bf16), W2[...].astype(bf16),
                           preferred_element_type=f32) + b2[...]


def _tc2(name_g, scal_g, seq_g, h4, wxs, whs, bs, dense):
    BB = 128
    grid = (B // BB,)
    time_spec = pl.BlockSpec((L, BB, D), lambda i: (0, i, 0))
    row_spec = pl.BlockSpec((BB, D), lambda i: (i, 0))

    def full(shape):
        n = len(shape)
        return pl.BlockSpec(shape, lambda i, n=n: (0,) * n)

    in_specs = ([time_spec] + [row_spec] * NSC + [time_spec] * NG
                + [pl.BlockSpec((BB, NO * D), lambda i: (i, 0))]
                + [full((D, 3 * D))] * NG + [full((D, 3 * D))] * NG
                + [full((3 * D,))] * NG
                + [full(d.shape) for d in dense])
    return pl.pallas_call(
        functools.partial(_tc2_body, BB),
        grid=grid,
        in_specs=in_specs,
        out_specs=pl.BlockSpec((BB, LAYER_SIZES[-1]), lambda i: (i, 0)),
        out_shape=jax.ShapeDtypeStruct((B, LAYER_SIZES[-1]), jnp.float32),
        scratch_shapes=[
            pltpu.VMEM((BB, 16 * D), jnp.float32),
            pltpu.VMEM((NG, L, BB, 3 * D), jnp.bfloat16),
        ],
    )(name_g, *scal_g, *seq_g, h4, *wxs, *whs, *bs, *dense)


def kernel(pl_name_tokens, pl_collaborative_idx, pl_pid_idx,
           duration_ms_seed_idx, n_songs_idx, n_artists_idx, n_albums_idx,
           artist_name_seq, track_uri_seq, track_name_seq,
           duration_ms_songs_seq, album_name_seq, artist_pop_seq,
           artists_followers_seq, track_pop_seq, artist_genres_seq, params):
    seq_by_name = {
        'artist_name': artist_name_seq, 'track_uri': track_uri_seq,
        'track_name': track_name_seq,
        'duration_ms_songs': duration_ms_songs_seq,
        'album_name': album_name_seq, 'artist_pop': artist_pop_seq,
        'artists_followers': artists_followers_seq,
        'track_pop': track_pop_seq, 'artist_genres': artist_genres_seq,
    }
    scal_idx = [pl_collaborative_idx, pl_pid_idx, duration_ms_seed_idx,
                n_songs_idx, n_artists_idx, n_albums_idx]

    big_names = ['pl_name'] + GATH_SEQ
    big_idx = [pl_name_tokens] + [seq_by_name[n] for n in GATH_SEQ]
    idxs = [jnp.reshape(jnp.swapaxes(a, 0, 1).astype(jnp.int32), (NW, RPW))
            for a in big_idx]
    idx_all = jnp.stack(idxs, axis=1)
    sidx = jnp.concatenate([jnp.reshape(a.astype(jnp.int32), (B // SCH, SCH))
                            for a in scal_idx], axis=0)
    tabs = [params['tab_' + n] for n in big_names]
    stabs = [params['tab_' + n] for n in SCALAR_FEATS]

    oh_idx = [jnp.swapaxes(seq_by_name[n], 0, 1).astype(jnp.int32)[:, :, None]
              for n in ONEHOT_SEQ]
    h4 = _tc1(oh_idx,
              [params['tab_' + n] for n in ONEHOT_SEQ],
              [params[n + '_Wx'] for n in ONEHOT_SEQ],
              [params[n + '_Wh'] for n in ONEHOT_SEQ],
              [params[n + '_b'] for n in ONEHOT_SEQ])

    g = _sc_gather(tabs, idx_all, stabs, sidx)
    big_g = [jnp.reshape(a, (L, B, D)) for a in g[:NBIG]]
    scal_g = [jnp.reshape(a, (B, D)) for a in g[NBIG:]]

    dense = [params['dense_W0'], params['dense_b0'],
             params['dense_W1'], params['dense_b1'],
             params['dense_W2'], params['dense_b2']]

    return _tc2(big_g[0], scal_g, big_g[1:], h4,
                [params[n + '_Wx'] for n in GATH_SEQ],
                [params[n + '_Wh'] for n in GATH_SEQ],
                [params[n + '_b'] for n in GATH_SEQ],
                dense)


# speedup vs baseline: 1.8485x; 1.0972x over previous
import functools

import jax
import jax.numpy as jnp
from jax import lax
from jax.experimental import pallas as pl
from jax.experimental.pallas import tpu as pltpu
from jax.experimental.pallas import tpu_sc as plsc

B = 1024
L = 20
D = 128
LAYER_SIZES = [512, 256, 128]
SCALAR_FEATS = ['pl_collaborative', 'pl_pid', 'duration_ms_seed', 'n_songs',
                'n_artists', 'n_albums']
SEQ_FEATS = ['artist_name', 'track_uri', 'track_name', 'duration_ms_songs',
             'album_name', 'artist_pop', 'artists_followers', 'track_pop',
             'artist_genres']
GATH_SEQ = ['artist_name', 'track_uri', 'track_name', 'album_name',
            'artist_genres']
ONEHOT_SEQ = ['duration_ms_songs', 'artist_pop', 'artists_followers',
              'track_pop']
ONEHOT_V = {'duration_ms_songs': 101, 'artist_pop': 11,
            'artists_followers': 11, 'track_pop': 11}
SLOT = {'pl_name': 0, 'pl_collaborative': 1, 'pl_pid': 2,
        'duration_ms_seed': 3, 'n_songs': 4, 'n_artists': 5, 'n_albums': 6,
        'artist_name': 7, 'track_uri': 8, 'track_name': 9,
        'duration_ms_songs': 10, 'album_name': 11, 'artist_pop': 12,
        'artists_followers': 13, 'track_pop': 14, 'artist_genres': 15}

NC = 2
NS = 16
NW = NC * NS
NBIG = 1 + len(GATH_SEQ)
ROWS_BIG = L * B
RPW = ROWS_BIG // NW
SCH = 256
NSC = 6
NSW = NSC * (B // SCH)
NG = len(GATH_SEQ)
NO = len(ONEHOT_SEQ)


def _sc_gather_body(*refs):
    tabs = refs[0:NBIG]
    idx_all = refs[NBIG]
    stabs = refs[NBIG + 1:NBIG + 1 + NSC]
    sidx = refs[NBIG + 1 + NSC]
    outs = refs[NBIG + 2 + NSC:2 * NBIG + 2 + NSC]
    souts = refs[2 * NBIG + 2 + NSC:2 * NBIG + 2 + 2 * NSC]
    rest = refs[2 * NBIG + 2 + 2 * NSC:]
    idx_fs = rest[0:NBIG]
    rows_v, idx_s, gsem, ssem, s2sem = rest[NBIG:]

    w = lax.axis_index("s") * NC + lax.axis_index("c")

    for f in range(NBIG):
        pltpu.sync_copy(idx_all.at[w, f], idx_fs[f])

    for f in range(NBIG):
        g = pltpu.make_async_copy(tabs[f].at[idx_fs[f]], rows_v, gsem)
        g.start()
        g.wait()
        s = pltpu.make_async_copy(rows_v, outs[f].at[w], ssem)
        s.start()
        s.wait()

    rows_sv = rows_v.at[pl.ds(0, SCH)]
    for f in range(NSC):
        for q in range(B // SCH):
            @pl.when(w == f * (B // SCH) + q)
            def _(f=f, q=q):
                pltpu.sync_copy(sidx.at[f * (B // SCH) + q], idx_s)
                g = pltpu.make_async_copy(stabs[f].at[idx_s], rows_sv, s2sem)
                g.start()
                g.wait()
                pltpu.sync_copy(rows_sv, souts[f].at[q])


def _sc_gather(tabs, idx_all, stabs, sidx):
    out_type = ([jax.ShapeDtypeStruct((NW, RPW, D), jnp.float32)
                 for _ in range(NBIG)]
                + [jax.ShapeDtypeStruct((B // SCH, SCH, D), jnp.float32)
                   for _ in range(NSC)])
    mesh = plsc.VectorSubcoreMesh(core_axis_name="c", subcore_axis_name="s")
    fn = pl.kernel(
        _sc_gather_body,
        out_type=out_type,
        mesh=mesh,
        scratch_types=(
            [pltpu.VMEM((RPW,), jnp.int32) for _ in range(NBIG)]
            + [
                pltpu.VMEM((RPW, D), jnp.float32),
                pltpu.VMEM((SCH,), jnp.int32),
                pltpu.SemaphoreType.DMA,
                pltpu.SemaphoreType.DMA,
                pltpu.SemaphoreType.DMA,
            ]
        ),
    )
    return fn(*tabs, idx_all, *stabs, sidx)


def _gru_loop(xz_ref, whg_b, whh_b, n, BB):
    f32 = jnp.float32
    bf16 = jnp.bfloat16

    def step(t, hs):
        new = []
        for f in range(n):
            h = hs[f]
            xt = xz_ref[f, t].astype(f32)
            hg = jnp.dot(h.astype(bf16), whg_b[f], preferred_element_type=f32)
            z = jax.nn.sigmoid(xt[:, :D] + hg[:, :D])
            r = jax.nn.sigmoid(xt[:, D:2 * D] + hg[:, D:])
            hh = jnp.tanh(xt[:, 2 * D:]
                          + jnp.dot((r * h).astype(bf16), whh_b[f],
                                    preferred_element_type=f32))
            new.append(z * h + (1.0 - z) * hh)
        return tuple(new)

    return lax.fori_loop(0, L, step,
                         tuple(jnp.zeros((BB, D), f32) for _ in range(n)))


def _tc1_body(BB, *refs):
    idxs = refs[0:NO]
    tabs = refs[NO:2 * NO]
    wxs = refs[2 * NO:3 * NO]
    whs = refs[3 * NO:4 * NO]
    bs = refs[4 * NO:5 * NO]
    out_ref = refs[5 * NO]
    xz_ref = refs[5 * NO + 1]

    f32 = jnp.float32
    bf16 = jnp.bfloat16
    whg_b = [whs[f][:, :2 * D].astype(bf16) for f in range(NO)]
    whh_b = [whs[f][:, 2 * D:].astype(bf16) for f in range(NO)]

    for f, name in enumerate(ONEHOT_SEQ):
        V = ONEHOT_V[name]
        tabwx = (jnp.dot(tabs[f][...], wxs[f][...],
                         preferred_element_type=f32)).astype(bf16)
        iv_all = idxs[f][...]
        for l in range(L):
            oh = (lax.broadcasted_iota(jnp.int32, (BB, V), 1)
                  == iv_all[:, l:l + 1]).astype(bf16)
            xz = jnp.dot(oh, tabwx, preferred_element_type=f32) + bs[f][...]
            xz_ref[f, l] = xz.astype(bf16)

    hs = _gru_loop(xz_ref, whg_b, whh_b, NO, BB)
    for f in range(NO):
        out_ref[:, f * D:(f + 1) * D] = hs[f]


def _tc1(idxs, tabs, wxs, whs, bs):
    BB = 128
    grid = (B // BB,)

    def full(shape):
        n = len(shape)
        return pl.BlockSpec(shape, lambda i, n=n: (0,) * n)

    in_specs = ([pl.BlockSpec((BB, L), lambda i: (i, 0))
                 for _ in range(NO)]
                + [full(t.shape) for t in tabs]
                + [full((D, 3 * D))] * NO
                + [full((D, 3 * D))] * NO
                + [full((3 * D,))] * NO)
    return pl.pallas_call(
        functools.partial(_tc1_body, BB),
        grid=grid,
        in_specs=in_specs,
        out_specs=pl.BlockSpec((BB, NO * D), lambda i: (i, 0)),
        out_shape=jax.ShapeDtypeStruct((B, NO * D), jnp.float32),
        scratch_shapes=[
            pltpu.VMEM((NO, L, BB, 3 * D), jnp.bfloat16),
        ],
    )(*idxs, *tabs, *wxs, *whs, *bs)


def _tc2_body(BB, *refs):
    name_ref = refs[0]
    scal = refs[1:1 + NSC]
    seqs = refs[1 + NSC:1 + NSC + NG]
    h4_ref = refs[1 + NSC + NG]
    wxs = refs[2 + NSC + NG:2 + NSC + 2 * NG]
    whs = refs[2 + NSC + 2 * NG:2 + NSC + 3 * NG]
    bs = refs[2 + NSC + 3 * NG:2 + NSC + 4 * NG]
    W0, b0, W1, b1, W2, b2 = refs[2 + NSC + 4 * NG:8 + NSC + 4 * NG]
    out_ref = refs[8 + NSC + 4 * NG]
    x_ref, xz_ref = refs[9 + NSC + 4 * NG:]

    f32 = jnp.float32
    bf16 = jnp.bfloat16

    x_ref[:, 0:D] = jnp.mean(name_ref[...], axis=0)
    for j in range(NSC):
        x_ref[:, (1 + j) * D:(2 + j) * D] = scal[j][...]
    h4 = h4_ref[...]
    x_ref[:, 10 * D:11 * D] = h4[:, 0:D]
    x_ref[:, 12 * D:15 * D] = h4[:, D:4 * D]

    whg_b = [whs[f][:, :2 * D].astype(bf16) for f in range(NG)]
    whh_b = [whs[f][:, 2 * D:].astype(bf16) for f in range(NG)]

    for f in range(NG):
        seq = jnp.reshape(seqs[f][...], (L * BB, D)).astype(bf16)
        xz = (jnp.dot(seq, wxs[f][...].astype(bf16),
                      preferred_element_type=f32) + bs[f][...])
        xz_ref[f] = jnp.reshape(xz, (L, BB, 3 * D)).astype(bf16)

    hs = _gru_loop(xz_ref, whg_b, whh_b, NG, BB)
    for f, name in enumerate(GATH_SEQ):
        s = SLOT[name]
        x_ref[:, s * D:(s + 1) * D] = hs[f]

    x = x_ref[...].astype(bf16)
    y = jax.nn.relu(jnp.dot(x, W0[...].astype(bf16),
                            preferred_element_type=f32) + b0[...])
    y = jax.nn.relu(jnp.dot(y.astype(bf16), W1[...].astype(bf16),
                            preferred_element_type=f32) + b1[...])
    out_ref[...] = jnp.dot(y.astype(bf16), W2[...].astype(bf16),
                           preferred_element_type=f32) + b2[...]


def _tc2(name_g, scal_g, seq_g, h4, wxs, whs, bs, dense):
    BB = 128
    grid = (B // BB,)
    time_spec = pl.BlockSpec((L, BB, D), lambda i: (0, i, 0))
    row_spec = pl.BlockSpec((BB, D), lambda i: (i, 0))

    def full(shape):
        n = len(shape)
        return pl.BlockSpec(shape, lambda i, n=n: (0,) * n)

    in_specs = ([time_spec] + [row_spec] * NSC + [time_spec] * NG
                + [pl.BlockSpec((BB, NO * D), lambda i: (i, 0))]
                + [full((D, 3 * D))] * NG + [full((D, 3 * D))] * NG
                + [full((3 * D,))] * NG
                + [full(d.shape) for d in dense])
    return pl.pallas_call(
        functools.partial(_tc2_body, BB),
        grid=grid,
        in_specs=in_specs,
        out_specs=pl.BlockSpec((BB, LAYER_SIZES[-1]), lambda i: (i, 0)),
        out_shape=jax.ShapeDtypeStruct((B, LAYER_SIZES[-1]), jnp.float32),
        scratch_shapes=[
            pltpu.VMEM((BB, 16 * D), jnp.float32),
            pltpu.VMEM((NG, L, BB, 3 * D), jnp.bfloat16),
        ],
    )(name_g, *scal_g, *seq_g, h4, *wxs, *whs, *bs, *dense)


def kernel(pl_name_tokens, pl_collaborative_idx, pl_pid_idx,
           duration_ms_seed_idx, n_songs_idx, n_artists_idx, n_albums_idx,
           artist_name_seq, track_uri_seq, track_name_seq,
           duration_ms_songs_seq, album_name_seq, artist_pop_seq,
           artists_followers_seq, track_pop_seq, artist_genres_seq, params):
    seq_by_name = {
        'artist_name': artist_name_seq, 'track_uri': track_uri_seq,
        'track_name': track_name_seq,
        'duration_ms_songs': duration_ms_songs_seq,
        'album_name': album_name_seq, 'artist_pop': artist_pop_seq,
        'artists_followers': artists_followers_seq,
        'track_pop': track_pop_seq, 'artist_genres': artist_genres_seq,
    }
    scal_idx = [pl_collaborative_idx, pl_pid_idx, duration_ms_seed_idx,
                n_songs_idx, n_artists_idx, n_albums_idx]

    big_names = ['pl_name'] + GATH_SEQ
    big_idx = [pl_name_tokens] + [seq_by_name[n] for n in GATH_SEQ]
    idxs = [jnp.reshape(jnp.swapaxes(a, 0, 1).astype(jnp.int32), (NW, RPW))
            for a in big_idx]
    idx_all = jnp.stack(idxs, axis=1)
    sidx = jnp.concatenate([jnp.reshape(a.astype(jnp.int32), (B // SCH, SCH))
                            for a in scal_idx], axis=0)
    tabs = [params['tab_' + n] for n in big_names]
    stabs = [params['tab_' + n] for n in SCALAR_FEATS]

    oh_idx = [seq_by_name[n].astype(jnp.int32) for n in ONEHOT_SEQ]
    h4 = _tc1(oh_idx,
              [params['tab_' + n] for n in ONEHOT_SEQ],
              [params[n + '_Wx'] for n in ONEHOT_SEQ],
              [params[n + '_Wh'] for n in ONEHOT_SEQ],
              [params[n + '_b'] for n in ONEHOT_SEQ])

    g = _sc_gather(tabs, idx_all, stabs, sidx)
    big_g = [jnp.reshape(a, (L, B, D)) for a in g[:NBIG]]
    scal_g = [jnp.reshape(a, (B, D)) for a in g[NBIG:]]

    dense = [params['dense_W0'], params['dense_b0'],
             params['dense_W1'], params['dense_b1'],
             params['dense_W2'], params['dense_b2']]

    return _tc2(big_g[0], scal_g, big_g[1:], h4,
                [params[n + '_Wx'] for n in GATH_SEQ],
                [params[n + '_Wh'] for n in GATH_SEQ],
                [params[n + '_b'] for n in GATH_SEQ],
                dense)


# speedup vs baseline: 1.8493x; 1.0005x over previous
import functools

import jax
import jax.numpy as jnp
from jax import lax
from jax.experimental import pallas as pl
from jax.experimental.pallas import tpu as pltpu
from jax.experimental.pallas import tpu_sc as plsc

B = 1024
L = 20
D = 128
LAYER_SIZES = [512, 256, 128]
SCALAR_FEATS = ['pl_collaborative', 'pl_pid', 'duration_ms_seed', 'n_songs',
                'n_artists', 'n_albums']
SEQ_FEATS = ['artist_name', 'track_uri', 'track_name', 'duration_ms_songs',
             'album_name', 'artist_pop', 'artists_followers', 'track_pop',
             'artist_genres']
GATH_SEQ = ['artist_name', 'track_uri', 'track_name', 'album_name',
            'artist_genres']
ONEHOT_SEQ = ['duration_ms_songs', 'artist_pop', 'artists_followers',
              'track_pop']
ONEHOT_V = {'duration_ms_songs': 101, 'artist_pop': 11,
            'artists_followers': 11, 'track_pop': 11}
SLOT = {'pl_name': 0, 'pl_collaborative': 1, 'pl_pid': 2,
        'duration_ms_seed': 3, 'n_songs': 4, 'n_artists': 5, 'n_albums': 6,
        'artist_name': 7, 'track_uri': 8, 'track_name': 9,
        'duration_ms_songs': 10, 'album_name': 11, 'artist_pop': 12,
        'artists_followers': 13, 'track_pop': 14, 'artist_genres': 15}

NC = 2
NS = 16
NW = NC * NS
NBIG = 1 + len(GATH_SEQ)
ROWS_BIG = L * B
RPW = ROWS_BIG // NW
SCH = 256
NSC = 6
NSW = NSC * (B // SCH)
NG = len(GATH_SEQ)
NO = len(ONEHOT_SEQ)


def _sc_gather_body(*refs):
    tabs = refs[0:NBIG]
    idx_all = refs[NBIG]
    stabs = refs[NBIG + 1:NBIG + 1 + NSC]
    sidx = refs[NBIG + 1 + NSC]
    outs = refs[NBIG + 2 + NSC:2 * NBIG + 2 + NSC]
    souts = refs[2 * NBIG + 2 + NSC:2 * NBIG + 2 + 2 * NSC]
    rest = refs[2 * NBIG + 2 + 2 * NSC:]
    idx_fs = rest[0:NBIG]
    rows_v, idx_s, gsem, ssem, s2sem = rest[NBIG:]

    w = lax.axis_index("s") * NC + lax.axis_index("c")

    for f in range(NBIG):
        pltpu.sync_copy(idx_all.at[w, f], idx_fs[f])

    for f in range(NBIG):
        g = pltpu.make_async_copy(tabs[f].at[idx_fs[f]], rows_v, gsem)
        g.start()
        g.wait()
        s = pltpu.make_async_copy(rows_v, outs[f].at[w], ssem)
        s.start()
        s.wait()

    rows_sv = rows_v.at[pl.ds(0, SCH)]
    for f in range(NSC):
        for q in range(B // SCH):
            @pl.when(w == f * (B // SCH) + q)
            def _(f=f, q=q):
                pltpu.sync_copy(sidx.at[f * (B // SCH) + q], idx_s)
                g = pltpu.make_async_copy(stabs[f].at[idx_s], rows_sv, s2sem)
                g.start()
                g.wait()
                pltpu.sync_copy(rows_sv, souts[f].at[q])


def _sc_gather(tabs, idx_all, stabs, sidx):
    out_type = ([jax.ShapeDtypeStruct((NW, RPW, D), jnp.float32)
                 for _ in range(NBIG)]
                + [jax.ShapeDtypeStruct((B // SCH, SCH, D), jnp.float32)
                   for _ in range(NSC)])
    mesh = plsc.VectorSubcoreMesh(core_axis_name="c", subcore_axis_name="s")
    fn = pl.kernel(
        _sc_gather_body,
        out_type=out_type,
        mesh=mesh,
        scratch_types=(
            [pltpu.VMEM((RPW,), jnp.int32) for _ in range(NBIG)]
            + [
                pltpu.VMEM((RPW, D), jnp.float32),
                pltpu.VMEM((SCH,), jnp.int32),
                pltpu.SemaphoreType.DMA,
                pltpu.SemaphoreType.DMA,
                pltpu.SemaphoreType.DMA,
            ]
        ),
    )
    return fn(*tabs, idx_all, *stabs, sidx)


def _gru_loop(xz_ref, whg_b, whh_b, n, BB):
    f32 = jnp.float32
    bf16 = jnp.bfloat16

    def step(t, hs):
        new = []
        for f in range(n):
            h = hs[f]
            xt = xz_ref[f, t]
            hg = jnp.dot(h.astype(bf16), whg_b[f], preferred_element_type=f32)
            z = jax.nn.sigmoid(xt[:, :D] + hg[:, :D])
            r = jax.nn.sigmoid(xt[:, D:2 * D] + hg[:, D:])
            hh = jnp.tanh(xt[:, 2 * D:]
                          + jnp.dot((r * h).astype(bf16), whh_b[f],
                                    preferred_element_type=f32))
            new.append(z * h + (1.0 - z) * hh)
        return tuple(new)

    return lax.fori_loop(0, L, step,
                         tuple(jnp.zeros((BB, D), f32) for _ in range(n)))


def _tc1_body(BB, *refs):
    idxs = refs[0:NO]
    tabs = refs[NO:2 * NO]
    wxs = refs[2 * NO:3 * NO]
    whs = refs[3 * NO:4 * NO]
    bs = refs[4 * NO:5 * NO]
    out_ref = refs[5 * NO]
    xz_ref = refs[5 * NO + 1]

    f32 = jnp.float32
    bf16 = jnp.bfloat16
    whg_b = [whs[f][:, :2 * D].astype(bf16) for f in range(NO)]
    whh_b = [whs[f][:, 2 * D:].astype(bf16) for f in range(NO)]

    for f, name in enumerate(ONEHOT_SEQ):
        V = ONEHOT_V[name]
        tabwx = (jnp.dot(tabs[f][...], wxs[f][...],
                         preferred_element_type=f32)).astype(bf16)
        iv_all = idxs[f][...]
        for l in range(L):
            oh = (lax.broadcasted_iota(jnp.int32, (BB, V), 1)
                  == iv_all[:, l:l + 1]).astype(bf16)
            xz = jnp.dot(oh, tabwx, preferred_element_type=f32) + bs[f][...]
            xz_ref[f, l] = xz

    hs = _gru_loop(xz_ref, whg_b, whh_b, NO, BB)
    for f in range(NO):
        out_ref[:, f * D:(f + 1) * D] = hs[f]


def _tc1(idxs, tabs, wxs, whs, bs):
    BB = 128
    grid = (B // BB,)

    def full(shape):
        n = len(shape)
        return pl.BlockSpec(shape, lambda i, n=n: (0,) * n)

    in_specs = ([pl.BlockSpec((BB, L), lambda i: (i, 0))
                 for _ in range(NO)]
                + [full(t.shape) for t in tabs]
                + [full((D, 3 * D))] * NO
                + [full((D, 3 * D))] * NO
                + [full((3 * D,))] * NO)
    return pl.pallas_call(
        functools.partial(_tc1_body, BB),
        grid=grid,
        in_specs=in_specs,
        out_specs=pl.BlockSpec((BB, NO * D), lambda i: (i, 0)),
        out_shape=jax.ShapeDtypeStruct((B, NO * D), jnp.float32),
        scratch_shapes=[
            pltpu.VMEM((NO, L, BB, 3 * D), jnp.float32),
        ],
    )(*idxs, *tabs, *wxs, *whs, *bs)


def _tc2_body(BB, *refs):
    name_ref = refs[0]
    scal = refs[1:1 + NSC]
    seqs = refs[1 + NSC:1 + NSC + NG]
    h4_ref = refs[1 + NSC + NG]
    wxs = refs[2 + NSC + NG:2 + NSC + 2 * NG]
    whs = refs[2 + NSC + 2 * NG:2 + NSC + 3 * NG]
    bs = refs[2 + NSC + 3 * NG:2 + NSC + 4 * NG]
    W0, b0, W1, b1, W2, b2 = refs[2 + NSC + 4 * NG:8 + NSC + 4 * NG]
    out_ref = refs[8 + NSC + 4 * NG]
    x_ref, xz_ref = refs[9 + NSC + 4 * NG:]

    f32 = jnp.float32
    bf16 = jnp.bfloat16

    x_ref[:, 0:D] = jnp.mean(name_ref[...], axis=0)
    for j in range(NSC):
        x_ref[:, (1 + j) * D:(2 + j) * D] = scal[j][...]
    h4 = h4_ref[...]
    x_ref[:, 10 * D:11 * D] = h4[:, 0:D]
    x_ref[:, 12 * D:15 * D] = h4[:, D:4 * D]

    whg_b = [whs[f][:, :2 * D].astype(bf16) for f in range(NG)]
    whh_b = [whs[f][:, 2 * D:].astype(bf16) for f in range(NG)]

    for f in range(NG):
        seq = jnp.reshape(seqs[f][...], (L * BB, D)).astype(bf16)
        xz = (jnp.dot(seq, wxs[f][...].astype(bf16),
                      preferred_element_type=f32) + bs[f][...])
        xz_ref[f] = jnp.reshape(xz, (L, BB, 3 * D))

    hs = _gru_loop(xz_ref, whg_b, whh_b, NG, BB)
    for f, name in enumerate(GATH_SEQ):
        s = SLOT[name]
        x_ref[:, s * D:(s + 1) * D] = hs[f]

    x = x_ref[...].astype(bf16)
    y = jax.nn.relu(jnp.dot(x, W0[...].astype(bf16),
                            preferred_element_type=f32) + b0[...])
    y = jax.nn.relu(jnp.dot(y.astype(bf16), W1[...].astype(bf16),
                            preferred_element_type=f32) + b1[...])
    out_ref[...] = jnp.dot(y.astype(bf16), W2[...].astype(bf16),
                           preferred_element_type=f32) + b2[...]


def _tc2(name_g, scal_g, seq_g, h4, wxs, whs, bs, dense):
    BB = 128
    grid = (B // BB,)
    time_spec = pl.BlockSpec((L, BB, D), lambda i: (0, i, 0))
    row_spec = pl.BlockSpec((BB, D), lambda i: (i, 0))

    def full(shape):
        n = len(shape)
        return pl.BlockSpec(shape, lambda i, n=n: (0,) * n)

    in_specs = ([time_spec] + [row_spec] * NSC + [time_spec] * NG
                + [pl.BlockSpec((BB, NO * D), lambda i: (i, 0))]
                + [full((D, 3 * D))] * NG + [full((D, 3 * D))] * NG
                + [full((3 * D,))] * NG
                + [full(d.shape) for d in dense])
    return pl.pallas_call(
        functools.partial(_tc2_body, BB),
        grid=grid,
        in_specs=in_specs,
        out_specs=pl.BlockSpec((BB, LAYER_SIZES[-1]), lambda i: (i, 0)),
        out_shape=jax.ShapeDtypeStruct((B, LAYER_SIZES[-1]), jnp.float32),
        scratch_shapes=[
            pltpu.VMEM((BB, 16 * D), jnp.float32),
            pltpu.VMEM((NG, L, BB, 3 * D), jnp.float32),
        ],
    )(name_g, *scal_g, *seq_g, h4, *wxs, *whs, *bs, *dense)


def kernel(pl_name_tokens, pl_collaborative_idx, pl_pid_idx,
           duration_ms_seed_idx, n_songs_idx, n_artists_idx, n_albums_idx,
           artist_name_seq, track_uri_seq, track_name_seq,
           duration_ms_songs_seq, album_name_seq, artist_pop_seq,
           artists_followers_seq, track_pop_seq, artist_genres_seq, params):
    seq_by_name = {
        'artist_name': artist_name_seq, 'track_uri': track_uri_seq,
        'track_name': track_name_seq,
        'duration_ms_songs': duration_ms_songs_seq,
        'album_name': album_name_seq, 'artist_pop': artist_pop_seq,
        'artists_followers': artists_followers_seq,
        'track_pop': track_pop_seq, 'artist_genres': artist_genres_seq,
    }
    scal_idx = [pl_collaborative_idx, pl_pid_idx, duration_ms_seed_idx,
                n_songs_idx, n_artists_idx, n_albums_idx]

    big_names = ['pl_name'] + GATH_SEQ
    big_idx = [pl_name_tokens] + [seq_by_name[n] for n in GATH_SEQ]
    idxs = [jnp.reshape(jnp.swapaxes(a, 0, 1).astype(jnp.int32), (NW, RPW))
            for a in big_idx]
    idx_all = jnp.stack(idxs, axis=1)
    sidx = jnp.concatenate([jnp.reshape(a.astype(jnp.int32), (B // SCH, SCH))
                            for a in scal_idx], axis=0)
    tabs = [params['tab_' + n] for n in big_names]
    stabs = [params['tab_' + n] for n in SCALAR_FEATS]

    oh_idx = [seq_by_name[n].astype(jnp.int32) for n in ONEHOT_SEQ]
    h4 = _tc1(oh_idx,
              [params['tab_' + n] for n in ONEHOT_SEQ],
              [params[n + '_Wx'] for n in ONEHOT_SEQ],
              [params[n + '_Wh'] for n in ONEHOT_SEQ],
              [params[n + '_b'] for n in ONEHOT_SEQ])

    g = _sc_gather(tabs, idx_all, stabs, sidx)
    big_g = [jnp.reshape(a, (L, B, D)) for a in g[:NBIG]]
    scal_g = [jnp.reshape(a, (B, D)) for a in g[NBIG:]]

    dense = [params['dense_W0'], params['dense_b0'],
             params['dense_W1'], params['dense_b1'],
             params['dense_W2'], params['dense_b2']]

    return _tc2(big_g[0], scal_g, big_g[1:], h4,
                [params[n + '_Wx'] for n in GATH_SEQ],
                [params[n + '_Wh'] for n in GATH_SEQ],
                [params[n + '_b'] for n in GATH_SEQ],
                dense)


# speedup vs baseline: 2.5796x; 1.3949x over previous
import functools

import jax
import jax.numpy as jnp
from jax import lax
from jax.experimental import pallas as pl
from jax.experimental.pallas import tpu as pltpu
from jax.experimental.pallas import tpu_sc as plsc

B = 1024
L = 20
D = 128
LAYER_SIZES = [512, 256, 128]
SCALAR_FEATS = ['pl_collaborative', 'pl_pid', 'duration_ms_seed', 'n_songs',
                'n_artists', 'n_albums']
SEQ_FEATS = ['artist_name', 'track_uri', 'track_name', 'duration_ms_songs',
             'album_name', 'artist_pop', 'artists_followers', 'track_pop',
             'artist_genres']
GATH_SEQ = ['artist_name', 'track_uri', 'track_name', 'album_name',
            'artist_genres']
ONEHOT_SEQ = ['duration_ms_songs', 'artist_pop', 'artists_followers',
              'track_pop']
ONEHOT_V = {'duration_ms_songs': 101, 'artist_pop': 11,
            'artists_followers': 11, 'track_pop': 11}
SLOT = {'pl_name': 0, 'pl_collaborative': 1, 'pl_pid': 2,
        'duration_ms_seed': 3, 'n_songs': 4, 'n_artists': 5, 'n_albums': 6,
        'artist_name': 7, 'track_uri': 8, 'track_name': 9,
        'duration_ms_songs': 10, 'album_name': 11, 'artist_pop': 12,
        'artists_followers': 13, 'track_pop': 14, 'artist_genres': 15}

NC = 2
NS = 16
NW = NC * NS
NBIG = 1 + len(GATH_SEQ)
ROWS_BIG = L * B
RPW = ROWS_BIG // NW
SCH = 256
NSC = 6
NSW = NSC * (B // SCH)
NG = len(GATH_SEQ)
NO = len(ONEHOT_SEQ)


def _sc_gather_body(*refs):
    tabs = refs[0:NBIG]
    idx_all = refs[NBIG]
    stabs = refs[NBIG + 1:NBIG + 1 + NSC]
    sidx = refs[NBIG + 1 + NSC]
    outs = refs[NBIG + 2 + NSC:2 * NBIG + 2 + NSC]
    souts = refs[2 * NBIG + 2 + NSC:2 * NBIG + 2 + 2 * NSC]
    rest = refs[2 * NBIG + 2 + 2 * NSC:]
    idx_fs = rest[0:NBIG]
    rows_v, idx_s, gsem, ssem, s2sem = rest[NBIG:]

    w = lax.axis_index("s") * NC + lax.axis_index("c")

    for f in range(NBIG):
        pltpu.sync_copy(idx_all.at[w, f], idx_fs[f])

    for f in range(NBIG):
        g = pltpu.make_async_copy(tabs[f].at[idx_fs[f]], rows_v, gsem)
        g.start()
        g.wait()
        s = pltpu.make_async_copy(rows_v, outs[f].at[w], ssem)
        s.start()
        s.wait()

    rows_sv = rows_v.at[pl.ds(0, SCH)]
    for f in range(NSC):
        for q in range(B // SCH):
            @pl.when(w == f * (B // SCH) + q)
            def _(f=f, q=q):
                pltpu.sync_copy(sidx.at[f * (B // SCH) + q], idx_s)
                g = pltpu.make_async_copy(stabs[f].at[idx_s], rows_sv, s2sem)
                g.start()
                g.wait()
                pltpu.sync_copy(rows_sv, souts[f].at[q])


def _sc_gather(tabs, idx_all, stabs, sidx):
    out_type = ([jax.ShapeDtypeStruct((NW, RPW, D), jnp.float32)
                 for _ in range(NBIG)]
                + [jax.ShapeDtypeStruct((B // SCH, SCH, D), jnp.float32)
                   for _ in range(NSC)])
    mesh = plsc.VectorSubcoreMesh(core_axis_name="c", subcore_axis_name="s")
    fn = pl.kernel(
        _sc_gather_body,
        out_type=out_type,
        mesh=mesh,
        scratch_types=(
            [pltpu.VMEM((RPW,), jnp.int32) for _ in range(NBIG)]
            + [
                pltpu.VMEM((RPW, D), jnp.float32),
                pltpu.VMEM((SCH,), jnp.int32),
                pltpu.SemaphoreType.DMA,
                pltpu.SemaphoreType.DMA,
                pltpu.SemaphoreType.DMA,
            ]
        ),
    )
    return fn(*tabs, idx_all, *stabs, sidx)


def _gru_loop(xz_ref, whg_b, whh_b, n, BB):
    f32 = jnp.float32
    bf16 = jnp.bfloat16

    def step(t, hs):
        new = []
        for f in range(n):
            h = hs[f]
            xt = xz_ref[f, t]
            hg = jnp.dot(h.astype(bf16), whg_b[f], preferred_element_type=f32)
            z = jax.nn.sigmoid(xt[:, :D] + hg[:, :D])
            r = jax.nn.sigmoid(xt[:, D:2 * D] + hg[:, D:])
            hh = jnp.tanh(xt[:, 2 * D:]
                          + jnp.dot((r * h).astype(bf16), whh_b[f],
                                    preferred_element_type=f32))
            new.append(z * h + (1.0 - z) * hh)
        return tuple(new)

    return lax.fori_loop(0, L, step,
                         tuple(jnp.zeros((BB, D), f32) for _ in range(n)))


def _tc1_body(BB, *refs):
    idxs = refs[0:NO]
    tabs = refs[NO:2 * NO]
    wxs = refs[2 * NO:3 * NO]
    whs = refs[3 * NO:4 * NO]
    bs = refs[4 * NO:5 * NO]
    out_ref = refs[5 * NO]
    xz_ref = refs[5 * NO + 1]

    f32 = jnp.float32
    bf16 = jnp.bfloat16
    whg_b = [whs[f][:, :2 * D].astype(bf16) for f in range(NO)]
    whh_b = [whs[f][:, 2 * D:].astype(bf16) for f in range(NO)]

    for f, name in enumerate(ONEHOT_SEQ):
        V = ONEHOT_V[name]
        tabwx = (jnp.dot(tabs[f][...], wxs[f][...],
                         preferred_element_type=f32)).astype(bf16)
        iv_all = idxs[f][...]
        for l in range(L):
            oh = (lax.broadcasted_iota(jnp.int32, (BB, V), 1)
                  == iv_all[:, l:l + 1]).astype(bf16)
            xz = jnp.dot(oh, tabwx, preferred_element_type=f32) + bs[f][...]
            xz_ref[f, l] = xz

    hs = _gru_loop(xz_ref, whg_b, whh_b, NO, BB)
    for f in range(NO):
        out_ref[:, f * D:(f + 1) * D] = hs[f]


def _tc1(idxs, tabs, wxs, whs, bs):
    BB = 256
    grid = (B // BB,)

    def full(shape):
        n = len(shape)
        return pl.BlockSpec(shape, lambda i, n=n: (0,) * n)

    in_specs = ([pl.BlockSpec((BB, L), lambda i: (i, 0))
                 for _ in range(NO)]
                + [full(t.shape) for t in tabs]
                + [full((D, 3 * D))] * NO
                + [full((D, 3 * D))] * NO
                + [full((3 * D,))] * NO)
    return pl.pallas_call(
        functools.partial(_tc1_body, BB),
        grid=grid,
        in_specs=in_specs,
        out_specs=pl.BlockSpec((BB, NO * D), lambda i: (i, 0)),
        out_shape=jax.ShapeDtypeStruct((B, NO * D), jnp.float32),
        scratch_shapes=[
            pltpu.VMEM((NO, L, BB, 3 * D), jnp.float32),
        ],
    )(*idxs, *tabs, *wxs, *whs, *bs)


def _tc2_body(BB, *refs):
    name_ref = refs[0]
    scal = refs[1:1 + NSC]
    seqs = refs[1 + NSC:1 + NSC + NG]
    h4_ref = refs[1 + NSC + NG]
    wxs = refs[2 + NSC + NG:2 + NSC + 2 * NG]
    whs = refs[2 + NSC + 2 * NG:2 + NSC + 3 * NG]
    bs = refs[2 + NSC + 3 * NG:2 + NSC + 4 * NG]
    W0, b0, W1, b1, W2, b2 = refs[2 + NSC + 4 * NG:8 + NSC + 4 * NG]
    out_ref = refs[8 + NSC + 4 * NG]
    (x_ref,) = refs[9 + NSC + 4 * NG:]

    f32 = jnp.float32
    bf16 = jnp.bfloat16

    x_ref[:, 0:D] = jnp.mean(name_ref[...], axis=0)
    for j in range(NSC):
        x_ref[:, (1 + j) * D:(2 + j) * D] = scal[j][...]
    h4 = h4_ref[...]
    x_ref[:, 10 * D:11 * D] = h4[:, 0:D]
    x_ref[:, 12 * D:15 * D] = h4[:, D:4 * D]

    whg_b = [whs[f][:, :2 * D].astype(bf16) for f in range(NG)]
    whh_b = [whs[f][:, 2 * D:].astype(bf16) for f in range(NG)]
    wx_b = [wxs[f][...].astype(bf16) for f in range(NG)]

    def step(t, hs):
        new = []
        for f in range(NG):
            h = hs[f]
            xt = (jnp.dot(seqs[f][t].astype(bf16), wx_b[f],
                          preferred_element_type=f32) + bs[f][...])
            hg = jnp.dot(h.astype(bf16), whg_b[f], preferred_element_type=f32)
            z = jax.nn.sigmoid(xt[:, :D] + hg[:, :D])
            r = jax.nn.sigmoid(xt[:, D:2 * D] + hg[:, D:])
            hh = jnp.tanh(xt[:, 2 * D:]
                          + jnp.dot((r * h).astype(bf16), whh_b[f],
                                    preferred_element_type=f32))
            new.append(z * h + (1.0 - z) * hh)
        return tuple(new)

    hs = lax.fori_loop(0, L, step,
                       tuple(jnp.zeros((BB, D), f32) for _ in range(NG)))
    for f, name in enumerate(GATH_SEQ):
        s = SLOT[name]
        x_ref[:, s * D:(s + 1) * D] = hs[f]

    x = x_ref[...].astype(bf16)
    y = jax.nn.relu(jnp.dot(x, W0[...].astype(bf16),
                            preferred_element_type=f32) + b0[...])
    y = jax.nn.relu(jnp.dot(y.astype(bf16), W1[...].astype(bf16),
                            preferred_element_type=f32) + b1[...])
    out_ref[...] = jnp.dot(y.astype(bf16), W2[...].astype(bf16),
                           preferred_element_type=f32) + b2[...]


def _tc2(name_g, scal_g, seq_g, h4, wxs, whs, bs, dense):
    BB = 256
    grid = (B // BB,)
    time_spec = pl.BlockSpec((L, BB, D), lambda i: (0, i, 0))
    row_spec = pl.BlockSpec((BB, D), lambda i: (i, 0))

    def full(shape):
        n = len(shape)
        return pl.BlockSpec(shape, lambda i, n=n: (0,) * n)

    in_specs = ([time_spec] + [row_spec] * NSC + [time_spec] * NG
                + [pl.BlockSpec((BB, NO * D), lambda i: (i, 0))]
                + [full((D, 3 * D))] * NG + [full((D, 3 * D))] * NG
                + [full((3 * D,))] * NG
                + [full(d.shape) for d in dense])
    return pl.pallas_call(
        functools.partial(_tc2_body, BB),
        grid=grid,
        in_specs=in_specs,
        out_specs=pl.BlockSpec((BB, LAYER_SIZES[-1]), lambda i: (i, 0)),
        out_shape=jax.ShapeDtypeStruct((B, LAYER_SIZES[-1]), jnp.float32),
        scratch_shapes=[
            pltpu.VMEM((BB, 16 * D), jnp.float32),
        ],
    )(name_g, *scal_g, *seq_g, h4, *wxs, *whs, *bs, *dense)


def kernel(pl_name_tokens, pl_collaborative_idx, pl_pid_idx,
           duration_ms_seed_idx, n_songs_idx, n_artists_idx, n_albums_idx,
           artist_name_seq, track_uri_seq, track_name_seq,
           duration_ms_songs_seq, album_name_seq, artist_pop_seq,
           artists_followers_seq, track_pop_seq, artist_genres_seq, params):
    seq_by_name = {
        'artist_name': artist_name_seq, 'track_uri': track_uri_seq,
        'track_name': track_name_seq,
        'duration_ms_songs': duration_ms_songs_seq,
        'album_name': album_name_seq, 'artist_pop': artist_pop_seq,
        'artists_followers': artists_followers_seq,
        'track_pop': track_pop_seq, 'artist_genres': artist_genres_seq,
    }
    scal_idx = [pl_collaborative_idx, pl_pid_idx, duration_ms_seed_idx,
                n_songs_idx, n_artists_idx, n_albums_idx]

    big_names = ['pl_name'] + GATH_SEQ
    big_idx = [pl_name_tokens] + [seq_by_name[n] for n in GATH_SEQ]
    idxs = [jnp.reshape(jnp.swapaxes(a, 0, 1).astype(jnp.int32), (NW, RPW))
            for a in big_idx]
    idx_all = jnp.stack(idxs, axis=1)
    sidx = jnp.concatenate([jnp.reshape(a.astype(jnp.int32), (B // SCH, SCH))
                            for a in scal_idx], axis=0)
    tabs = [params['tab_' + n] for n in big_names]
    stabs = [params['tab_' + n] for n in SCALAR_FEATS]

    oh_idx = [seq_by_name[n].astype(jnp.int32) for n in ONEHOT_SEQ]
    h4 = _tc1(oh_idx,
              [params['tab_' + n] for n in ONEHOT_SEQ],
              [params[n + '_Wx'] for n in ONEHOT_SEQ],
              [params[n + '_Wh'] for n in ONEHOT_SEQ],
              [params[n + '_b'] for n in ONEHOT_SEQ])

    g = _sc_gather(tabs, idx_all, stabs, sidx)
    big_g = [jnp.reshape(a, (L, B, D)) for a in g[:NBIG]]
    scal_g = [jnp.reshape(a, (B, D)) for a in g[NBIG:]]

    dense = [params['dense_W0'], params['dense_b0'],
             params['dense_W1'], params['dense_b1'],
             params['dense_W2'], params['dense_b2']]

    return _tc2(big_g[0], scal_g, big_g[1:], h4,
                [params[n + '_Wx'] for n in GATH_SEQ],
                [params[n + '_Wh'] for n in GATH_SEQ],
                [params[n + '_b'] for n in GATH_SEQ],
                dense)


# speedup vs baseline: 3.0462x; 1.1809x over previous
import functools

import jax
import jax.numpy as jnp
from jax import lax
from jax.experimental import pallas as pl
from jax.experimental.pallas import tpu as pltpu
from jax.experimental.pallas import tpu_sc as plsc

B = 1024
L = 20
D = 128
LAYER_SIZES = [512, 256, 128]
SCALAR_FEATS = ['pl_collaborative', 'pl_pid', 'duration_ms_seed', 'n_songs',
                'n_artists', 'n_albums']
SEQ_FEATS = ['artist_name', 'track_uri', 'track_name', 'duration_ms_songs',
             'album_name', 'artist_pop', 'artists_followers', 'track_pop',
             'artist_genres']
GATH_SEQ = ['artist_name', 'track_uri', 'track_name', 'album_name',
            'artist_genres']
ONEHOT_SEQ = ['duration_ms_songs', 'artist_pop', 'artists_followers',
              'track_pop']
ONEHOT_V = {'duration_ms_songs': 101, 'artist_pop': 11,
            'artists_followers': 11, 'track_pop': 11}
SLOT = {'pl_name': 0, 'pl_collaborative': 1, 'pl_pid': 2,
        'duration_ms_seed': 3, 'n_songs': 4, 'n_artists': 5, 'n_albums': 6,
        'artist_name': 7, 'track_uri': 8, 'track_name': 9,
        'duration_ms_songs': 10, 'album_name': 11, 'artist_pop': 12,
        'artists_followers': 13, 'track_pop': 14, 'artist_genres': 15}

NC = 2
NS = 16
NW = NC * NS
NBIG = 1 + len(GATH_SEQ)
ROWS_BIG = L * B
RPW = ROWS_BIG // NW
SCH = 256
NSC = 6
NSW = NSC * (B // SCH)
NG = len(GATH_SEQ)
NO = len(ONEHOT_SEQ)


def _sc_gather_body(*refs):
    tabs = refs[0:NBIG]
    idx_all = refs[NBIG]
    stabs = refs[NBIG + 1:NBIG + 1 + NSC]
    sidx = refs[NBIG + 1 + NSC]
    outs = refs[NBIG + 2 + NSC:2 * NBIG + 2 + NSC]
    souts = refs[2 * NBIG + 2 + NSC:2 * NBIG + 2 + 2 * NSC]
    rest = refs[2 * NBIG + 2 + 2 * NSC:]
    idx_fs = rest[0:NBIG]
    rows_v, idx_s, gsem, ssem, s2sem = rest[NBIG:]

    w = lax.axis_index("s") * NC + lax.axis_index("c")

    for f in range(NBIG):
        pltpu.sync_copy(idx_all.at[w, f], idx_fs[f])

    for f in range(NBIG):
        g = pltpu.make_async_copy(tabs[f].at[idx_fs[f]], rows_v, gsem)
        g.start()
        g.wait()
        s = pltpu.make_async_copy(rows_v, outs[f].at[w], ssem)
        s.start()
        s.wait()

    rows_sv = rows_v.at[pl.ds(0, SCH)]
    for f in range(NSC):
        for q in range(B // SCH):
            @pl.when(w == f * (B // SCH) + q)
            def _(f=f, q=q):
                pltpu.sync_copy(sidx.at[f * (B // SCH) + q], idx_s)
                g = pltpu.make_async_copy(stabs[f].at[idx_s], rows_sv, s2sem)
                g.start()
                g.wait()
                pltpu.sync_copy(rows_sv, souts[f].at[q])


def _sc_gather(tabs, idx_all, stabs, sidx):
    out_type = ([jax.ShapeDtypeStruct((NW, RPW, D), jnp.float32)
                 for _ in range(NBIG)]
                + [jax.ShapeDtypeStruct((B // SCH, SCH, D), jnp.float32)
                   for _ in range(NSC)])
    mesh = plsc.VectorSubcoreMesh(core_axis_name="c", subcore_axis_name="s")
    fn = pl.kernel(
        _sc_gather_body,
        out_type=out_type,
        mesh=mesh,
        scratch_types=(
            [pltpu.VMEM((RPW,), jnp.int32) for _ in range(NBIG)]
            + [
                pltpu.VMEM((RPW, D), jnp.float32),
                pltpu.VMEM((SCH,), jnp.int32),
                pltpu.SemaphoreType.DMA,
                pltpu.SemaphoreType.DMA,
                pltpu.SemaphoreType.DMA,
            ]
        ),
    )
    return fn(*tabs, idx_all, *stabs, sidx)


def _gru_loop(xz_ref, whg_b, whh_b, n, BB):
    f32 = jnp.float32
    bf16 = jnp.bfloat16

    def step(t, hs):
        new = []
        for f in range(n):
            h = hs[f]
            xt = xz_ref[f, t]
            hg = jnp.dot(h.astype(bf16), whg_b[f], preferred_element_type=f32)
            z = jax.nn.sigmoid(xt[:, :D] + hg[:, :D])
            r = jax.nn.sigmoid(xt[:, D:2 * D] + hg[:, D:])
            hh = jnp.tanh(xt[:, 2 * D:]
                          + jnp.dot((r * h).astype(bf16), whh_b[f],
                                    preferred_element_type=f32))
            new.append(z * h + (1.0 - z) * hh)
        return tuple(new)

    return lax.fori_loop(0, L, step,
                         tuple(jnp.zeros((BB, D), f32) for _ in range(n)))


def _tc1_body(BB, *refs):
    idxs = refs[0:NO]
    tabs = refs[NO:2 * NO]
    wxs = refs[2 * NO:3 * NO]
    whs = refs[3 * NO:4 * NO]
    bs = refs[4 * NO:5 * NO]
    out_ref = refs[5 * NO]
    xz_ref = refs[5 * NO + 1]

    f32 = jnp.float32
    bf16 = jnp.bfloat16
    whg_b = [whs[f][:, :2 * D].astype(bf16) for f in range(NO)]
    whh_b = [whs[f][:, 2 * D:].astype(bf16) for f in range(NO)]

    for f, name in enumerate(ONEHOT_SEQ):
        V = ONEHOT_V[name]
        tabwx = (jnp.dot(tabs[f][...], wxs[f][...],
                         preferred_element_type=f32)).astype(bf16)
        iv_all = idxs[f][...]
        for l in range(L):
            oh = (lax.broadcasted_iota(jnp.int32, (BB, V), 1)
                  == iv_all[:, l:l + 1]).astype(bf16)
            xz = jnp.dot(oh, tabwx, preferred_element_type=f32) + bs[f][...]
            xz_ref[f, l] = xz

    hs = _gru_loop(xz_ref, whg_b, whh_b, NO, BB)
    for f in range(NO):
        out_ref[:, f * D:(f + 1) * D] = hs[f]


def _tc1(idxs, tabs, wxs, whs, bs):
    BB = 256
    grid = (B // BB,)

    def full(shape):
        n = len(shape)
        return pl.BlockSpec(shape, lambda i, n=n: (0,) * n)

    in_specs = ([pl.BlockSpec((BB, L), lambda i: (i, 0))
                 for _ in range(NO)]
                + [full(t.shape) for t in tabs]
                + [full((D, 3 * D))] * NO
                + [full((D, 3 * D))] * NO
                + [full((3 * D,))] * NO)
    return pl.pallas_call(
        functools.partial(_tc1_body, BB),
        grid=grid,
        in_specs=in_specs,
        out_specs=pl.BlockSpec((BB, NO * D), lambda i: (i, 0)),
        out_shape=jax.ShapeDtypeStruct((B, NO * D), jnp.float32),
        scratch_shapes=[
            pltpu.VMEM((NO, L, BB, 3 * D), jnp.float32),
        ],
    )(*idxs, *tabs, *wxs, *whs, *bs)


def _tc2_body(BB, *refs):
    name_ref = refs[0]
    scal = refs[1:1 + NSC]
    seqs = refs[1 + NSC:1 + NSC + NG]
    h4_ref = refs[1 + NSC + NG]
    wxs = refs[2 + NSC + NG:2 + NSC + 2 * NG]
    whs = refs[2 + NSC + 2 * NG:2 + NSC + 3 * NG]
    bs = refs[2 + NSC + 3 * NG:2 + NSC + 4 * NG]
    W0, b0, W1, b1, W2, b2 = refs[2 + NSC + 4 * NG:8 + NSC + 4 * NG]
    out_ref = refs[8 + NSC + 4 * NG]
    (x_ref,) = refs[9 + NSC + 4 * NG:]

    f32 = jnp.float32
    bf16 = jnp.bfloat16

    x_ref[:, 0:D] = jnp.mean(name_ref[...], axis=0)
    for j in range(NSC):
        x_ref[:, (1 + j) * D:(2 + j) * D] = scal[j][...]
    h4 = h4_ref[...]
    x_ref[:, 10 * D:11 * D] = h4[:, 0:D]
    x_ref[:, 12 * D:15 * D] = h4[:, D:4 * D]

    whh_b = [whs[f][:, 2 * D:].astype(bf16) for f in range(NG)]
    wc_b = []
    for f in range(NG):
        whg_pad = jnp.concatenate(
            [whs[f][:, :2 * D], jnp.zeros((D, D), f32)], axis=1)
        wc_b.append(jnp.concatenate([wxs[f][...], whg_pad],
                                    axis=0).astype(bf16))

    def step(t, hs):
        new = []
        for f in range(NG):
            h = hs[f]
            xin = jnp.concatenate([seqs[f][t], h], axis=1).astype(bf16)
            comb = (jnp.dot(xin, wc_b[f], preferred_element_type=f32)
                    + bs[f][...])
            z = jax.nn.sigmoid(comb[:, :D])
            r = jax.nn.sigmoid(comb[:, D:2 * D])
            hh = jnp.tanh(comb[:, 2 * D:]
                          + jnp.dot((r * h).astype(bf16), whh_b[f],
                                    preferred_element_type=f32))
            new.append(z * h + (1.0 - z) * hh)
        return tuple(new)

    hs = lax.fori_loop(0, L, step,
                       tuple(jnp.zeros((BB, D), f32) for _ in range(NG)))
    for f, name in enumerate(GATH_SEQ):
        s = SLOT[name]
        x_ref[:, s * D:(s + 1) * D] = hs[f]

    x = x_ref[...].astype(bf16)
    y = jax.nn.relu(jnp.dot(x, W0[...].astype(bf16),
                            preferred_element_type=f32) + b0[...])
    y = jax.nn.relu(jnp.dot(y.astype(bf16), W1[...].astype(bf16),
                            preferred_element_type=f32) + b1[...])
    out_ref[...] = jnp.dot(y.astype(bf16), W2[...].astype(bf16),
                           preferred_element_type=f32) + b2[...]


def _tc2(name_g, scal_g, seq_g, h4, wxs, whs, bs, dense):
    BB = 256
    grid = (B // BB,)
    time_spec = pl.BlockSpec((L, BB, D), lambda i: (0, i, 0))
    row_spec = pl.BlockSpec((BB, D), lambda i: (i, 0))

    def full(shape):
        n = len(shape)
        return pl.BlockSpec(shape, lambda i, n=n: (0,) * n)

    in_specs = ([time_spec] + [row_spec] * NSC + [time_spec] * NG
                + [pl.BlockSpec((BB, NO * D), lambda i: (i, 0))]
                + [full((D, 3 * D))] * NG + [full((D, 3 * D))] * NG
                + [full((3 * D,))] * NG
                + [full(d.shape) for d in dense])
    return pl.pallas_call(
        functools.partial(_tc2_body, BB),
        grid=grid,
        in_specs=in_specs,
        out_specs=pl.BlockSpec((BB, LAYER_SIZES[-1]), lambda i: (i, 0)),
        out_shape=jax.ShapeDtypeStruct((B, LAYER_SIZES[-1]), jnp.float32),
        scratch_shapes=[
            pltpu.VMEM((BB, 16 * D), jnp.float32),
        ],
    )(name_g, *scal_g, *seq_g, h4, *wxs, *whs, *bs, *dense)


def kernel(pl_name_tokens, pl_collaborative_idx, pl_pid_idx,
           duration_ms_seed_idx, n_songs_idx, n_artists_idx, n_albums_idx,
           artist_name_seq, track_uri_seq, track_name_seq,
           duration_ms_songs_seq, album_name_seq, artist_pop_seq,
           artists_followers_seq, track_pop_seq, artist_genres_seq, params):
    seq_by_name = {
        'artist_name': artist_name_seq, 'track_uri': track_uri_seq,
        'track_name': track_name_seq,
        'duration_ms_songs': duration_ms_songs_seq,
        'album_name': album_name_seq, 'artist_pop': artist_pop_seq,
        'artists_followers': artists_followers_seq,
        'track_pop': track_pop_seq, 'artist_genres': artist_genres_seq,
    }
    scal_idx = [pl_collaborative_idx, pl_pid_idx, duration_ms_seed_idx,
                n_songs_idx, n_artists_idx, n_albums_idx]

    big_names = ['pl_name'] + GATH_SEQ
    big_idx = [pl_name_tokens] + [seq_by_name[n] for n in GATH_SEQ]
    idxs = [jnp.reshape(jnp.swapaxes(a, 0, 1).astype(jnp.int32), (NW, RPW))
            for a in big_idx]
    idx_all = jnp.stack(idxs, axis=1)
    sidx = jnp.concatenate([jnp.reshape(a.astype(jnp.int32), (B // SCH, SCH))
                            for a in scal_idx], axis=0)
    tabs = [params['tab_' + n] for n in big_names]
    stabs = [params['tab_' + n] for n in SCALAR_FEATS]

    oh_idx = [seq_by_name[n].astype(jnp.int32) for n in ONEHOT_SEQ]
    h4 = _tc1(oh_idx,
              [params['tab_' + n] for n in ONEHOT_SEQ],
              [params[n + '_Wx'] for n in ONEHOT_SEQ],
              [params[n + '_Wh'] for n in ONEHOT_SEQ],
              [params[n + '_b'] for n in ONEHOT_SEQ])

    g = _sc_gather(tabs, idx_all, stabs, sidx)
    big_g = [jnp.reshape(a, (L, B, D)) for a in g[:NBIG]]
    scal_g = [jnp.reshape(a, (B, D)) for a in g[NBIG:]]

    dense = [params['dense_W0'], params['dense_b0'],
             params['dense_W1'], params['dense_b1'],
             params['dense_W2'], params['dense_b2']]

    return _tc2(big_g[0], scal_g, big_g[1:], h4,
                [params[n + '_Wx'] for n in GATH_SEQ],
                [params[n + '_Wh'] for n in GATH_SEQ],
                [params[n + '_b'] for n in GATH_SEQ],
                dense)


# speedup vs baseline: 3.5716x; 1.1725x over previous
import functools

import jax
import jax.numpy as jnp
from jax import lax
from jax.experimental import pallas as pl
from jax.experimental.pallas import tpu as pltpu
from jax.experimental.pallas import tpu_sc as plsc

B = 1024
L = 20
D = 128
LAYER_SIZES = [512, 256, 128]
SCALAR_FEATS = ['pl_collaborative', 'pl_pid', 'duration_ms_seed', 'n_songs',
                'n_artists', 'n_albums']
SEQ_FEATS = ['artist_name', 'track_uri', 'track_name', 'duration_ms_songs',
             'album_name', 'artist_pop', 'artists_followers', 'track_pop',
             'artist_genres']
GATH_SEQ = ['artist_name', 'track_uri', 'track_name', 'album_name',
            'artist_genres']
ONEHOT_SEQ = ['duration_ms_songs', 'artist_pop', 'artists_followers',
              'track_pop']
ONEHOT_V = {'duration_ms_songs': 101, 'artist_pop': 11,
            'artists_followers': 11, 'track_pop': 11}
SLOT = {'pl_name': 0, 'pl_collaborative': 1, 'pl_pid': 2,
        'duration_ms_seed': 3, 'n_songs': 4, 'n_artists': 5, 'n_albums': 6,
        'artist_name': 7, 'track_uri': 8, 'track_name': 9,
        'duration_ms_songs': 10, 'album_name': 11, 'artist_pop': 12,
        'artists_followers': 13, 'track_pop': 14, 'artist_genres': 15}

NC = 2
NS = 16
NW = NC * NS
NBIG = 1 + len(GATH_SEQ)
ROWS_BIG = L * B
RPW = ROWS_BIG // NW
SCH = 256
NSC = 6
NSW = NSC * (B // SCH)
NG = len(GATH_SEQ)
NO = len(ONEHOT_SEQ)


def _sc_gather_body(*refs):
    tabs = refs[0:NBIG]
    idx_all = refs[NBIG]
    stabs = refs[NBIG + 1:NBIG + 1 + NSC]
    sidx = refs[NBIG + 1 + NSC]
    outs = refs[NBIG + 2 + NSC:2 * NBIG + 2 + NSC]
    souts = refs[2 * NBIG + 2 + NSC:2 * NBIG + 2 + 2 * NSC]
    rest = refs[2 * NBIG + 2 + 2 * NSC:]
    idx_fs = rest[0:NBIG]
    rows_v, idx_s, gsem, ssem, s2sem = rest[NBIG:]

    w = lax.axis_index("s") * NC + lax.axis_index("c")

    for f in range(NBIG):
        pltpu.sync_copy(idx_all.at[w, f], idx_fs[f])

    for f in range(NBIG):
        g = pltpu.make_async_copy(tabs[f].at[idx_fs[f]], rows_v, gsem)
        g.start()
        g.wait()
        s = pltpu.make_async_copy(rows_v, outs[f].at[w], ssem)
        s.start()
        s.wait()

    rows_sv = rows_v.at[pl.ds(0, SCH)]
    for f in range(NSC):
        for q in range(B // SCH):
            @pl.when(w == f * (B // SCH) + q)
            def _(f=f, q=q):
                pltpu.sync_copy(sidx.at[f * (B // SCH) + q], idx_s)
                g = pltpu.make_async_copy(stabs[f].at[idx_s], rows_sv, s2sem)
                g.start()
                g.wait()
                pltpu.sync_copy(rows_sv, souts[f].at[q])


def _sc_gather(tabs, idx_all, stabs, sidx):
    out_type = ([jax.ShapeDtypeStruct((NW, RPW, D), jnp.float32)
                 for _ in range(NBIG)]
                + [jax.ShapeDtypeStruct((B // SCH, SCH, D), jnp.float32)
                   for _ in range(NSC)])
    mesh = plsc.VectorSubcoreMesh(core_axis_name="c", subcore_axis_name="s")
    fn = pl.kernel(
        _sc_gather_body,
        out_type=out_type,
        mesh=mesh,
        scratch_types=(
            [pltpu.VMEM((RPW,), jnp.int32) for _ in range(NBIG)]
            + [
                pltpu.VMEM((RPW, D), jnp.float32),
                pltpu.VMEM((SCH,), jnp.int32),
                pltpu.SemaphoreType.DMA,
                pltpu.SemaphoreType.DMA,
                pltpu.SemaphoreType.DMA,
            ]
        ),
    )
    return fn(*tabs, idx_all, *stabs, sidx)


def _gru_loop(xz_ref, whg_b, whh_b, n, BB):
    f32 = jnp.float32
    bf16 = jnp.bfloat16

    def step(t, hs):
        new = []
        for f in range(n):
            h = hs[f]
            xt = xz_ref[f, t]
            hg = jnp.dot(h.astype(bf16), whg_b[f], preferred_element_type=f32)
            z = jax.nn.sigmoid(xt[:, :D] + hg[:, :D])
            r = jax.nn.sigmoid(xt[:, D:2 * D] + hg[:, D:])
            hh = jnp.tanh(xt[:, 2 * D:]
                          + jnp.dot((r * h).astype(bf16), whh_b[f],
                                    preferred_element_type=f32))
            new.append(z * h + (1.0 - z) * hh)
        return tuple(new)

    return lax.fori_loop(0, L, step,
                         tuple(jnp.zeros((BB, D), f32) for _ in range(n)))


def _tc1_body(BB, *refs):
    idxs = refs[0:NO]
    tabs = refs[NO:2 * NO]
    wxs = refs[2 * NO:3 * NO]
    whs = refs[3 * NO:4 * NO]
    bs = refs[4 * NO:5 * NO]
    out_ref = refs[5 * NO]

    f32 = jnp.float32
    bf16 = jnp.bfloat16
    whh_b = [whs[f][:, 2 * D:].astype(bf16) for f in range(NO)]
    wc_b = []
    for f in range(NO):
        tabwx = jnp.dot(tabs[f][...], wxs[f][...], preferred_element_type=f32)
        whg_pad = jnp.concatenate(
            [whs[f][:, :2 * D], jnp.zeros((D, D), f32)], axis=1)
        wc_b.append(jnp.concatenate([tabwx, whg_pad], axis=0).astype(bf16))
    ivs = [idxs[f][...] for f in range(NO)]

    hs = [jnp.zeros((BB, D), f32) for _ in range(NO)]
    for t in range(L):
        new = []
        for f, name in enumerate(ONEHOT_SEQ):
            V = ONEHOT_V[name]
            h = hs[f]
            oh = (lax.broadcasted_iota(jnp.int32, (BB, V), 1)
                  == ivs[f][:, t:t + 1]).astype(bf16)
            xin = jnp.concatenate([oh, h.astype(bf16)], axis=1)
            comb = (jnp.dot(xin, wc_b[f], preferred_element_type=f32)
                    + bs[f][...])
            z = jax.nn.sigmoid(comb[:, :D])
            r = jax.nn.sigmoid(comb[:, D:2 * D])
            hh = jnp.tanh(comb[:, 2 * D:]
                          + jnp.dot((r * h).astype(bf16), whh_b[f],
                                    preferred_element_type=f32))
            new.append(z * h + (1.0 - z) * hh)
        hs = new

    for f in range(NO):
        out_ref[:, f * D:(f + 1) * D] = hs[f]


def _tc1(idxs, tabs, wxs, whs, bs):
    BB = 256
    grid = (B // BB,)

    def full(shape):
        n = len(shape)
        return pl.BlockSpec(shape, lambda i, n=n: (0,) * n)

    in_specs = ([pl.BlockSpec((BB, L), lambda i: (i, 0))
                 for _ in range(NO)]
                + [full(t.shape) for t in tabs]
                + [full((D, 3 * D))] * NO
                + [full((D, 3 * D))] * NO
                + [full((3 * D,))] * NO)
    return pl.pallas_call(
        functools.partial(_tc1_body, BB),
        grid=grid,
        in_specs=in_specs,
        out_specs=pl.BlockSpec((BB, NO * D), lambda i: (i, 0)),
        out_shape=jax.ShapeDtypeStruct((B, NO * D), jnp.float32),
    )(*idxs, *tabs, *wxs, *whs, *bs)


def _tc2_body(BB, *refs):
    name_ref = refs[0]
    scal = refs[1:1 + NSC]
    seqs = refs[1 + NSC:1 + NSC + NG]
    h4_ref = refs[1 + NSC + NG]
    wxs = refs[2 + NSC + NG:2 + NSC + 2 * NG]
    whs = refs[2 + NSC + 2 * NG:2 + NSC + 3 * NG]
    bs = refs[2 + NSC + 3 * NG:2 + NSC + 4 * NG]
    W0, b0, W1, b1, W2, b2 = refs[2 + NSC + 4 * NG:8 + NSC + 4 * NG]
    out_ref = refs[8 + NSC + 4 * NG]
    (x_ref,) = refs[9 + NSC + 4 * NG:]

    f32 = jnp.float32
    bf16 = jnp.bfloat16

    x_ref[:, 0:D] = jnp.mean(name_ref[...], axis=0)
    for j in range(NSC):
        x_ref[:, (1 + j) * D:(2 + j) * D] = scal[j][...]
    h4 = h4_ref[...]
    x_ref[:, 10 * D:11 * D] = h4[:, 0:D]
    x_ref[:, 12 * D:15 * D] = h4[:, D:4 * D]

    whh_b = [whs[f][:, 2 * D:].astype(bf16) for f in range(NG)]
    wc_b = []
    for f in range(NG):
        whg_pad = jnp.concatenate(
            [whs[f][:, :2 * D], jnp.zeros((D, D), f32)], axis=1)
        wc_b.append(jnp.concatenate([wxs[f][...], whg_pad],
                                    axis=0).astype(bf16))

    def step(t, hs):
        new = []
        for f in range(NG):
            h = hs[f]
            xin = jnp.concatenate([seqs[f][t], h], axis=1).astype(bf16)
            comb = (jnp.dot(xin, wc_b[f], preferred_element_type=f32)
                    + bs[f][...])
            z = jax.nn.sigmoid(comb[:, :D])
            r = jax.nn.sigmoid(comb[:, D:2 * D])
            hh = jnp.tanh(comb[:, 2 * D:]
                          + jnp.dot((r * h).astype(bf16), whh_b[f],
                                    preferred_element_type=f32))
            new.append(z * h + (1.0 - z) * hh)
        return tuple(new)

    hs = lax.fori_loop(0, L, step,
                       tuple(jnp.zeros((BB, D), f32) for _ in range(NG)))
    for f, name in enumerate(GATH_SEQ):
        s = SLOT[name]
        x_ref[:, s * D:(s + 1) * D] = hs[f]

    x = x_ref[...].astype(bf16)
    y = jax.nn.relu(jnp.dot(x, W0[...].astype(bf16),
                            preferred_element_type=f32) + b0[...])
    y = jax.nn.relu(jnp.dot(y.astype(bf16), W1[...].astype(bf16),
                            preferred_element_type=f32) + b1[...])
    out_ref[...] = jnp.dot(y.astype(bf16), W2[...].astype(bf16),
                           preferred_element_type=f32) + b2[...]


def _tc2(name_g, scal_g, seq_g, h4, wxs, whs, bs, dense):
    BB = 256
    grid = (B // BB,)
    time_spec = pl.BlockSpec((L, BB, D), lambda i: (0, i, 0))
    row_spec = pl.BlockSpec((BB, D), lambda i: (i, 0))

    def full(shape):
        n = len(shape)
        return pl.BlockSpec(shape, lambda i, n=n: (0,) * n)

    in_specs = ([time_spec] + [row_spec] * NSC + [time_spec] * NG
                + [pl.BlockSpec((BB, NO * D), lambda i: (i, 0))]
                + [full((D, 3 * D))] * NG + [full((D, 3 * D))] * NG
                + [full((3 * D,))] * NG
                + [full(d.shape) for d in dense])
    return pl.pallas_call(
        functools.partial(_tc2_body, BB),
        grid=grid,
        in_specs=in_specs,
        out_specs=pl.BlockSpec((BB, LAYER_SIZES[-1]), lambda i: (i, 0)),
        out_shape=jax.ShapeDtypeStruct((B, LAYER_SIZES[-1]), jnp.float32),
        scratch_shapes=[
            pltpu.VMEM((BB, 16 * D), jnp.float32),
        ],
    )(name_g, *scal_g, *seq_g, h4, *wxs, *whs, *bs, *dense)


def kernel(pl_name_tokens, pl_collaborative_idx, pl_pid_idx,
           duration_ms_seed_idx, n_songs_idx, n_artists_idx, n_albums_idx,
           artist_name_seq, track_uri_seq, track_name_seq,
           duration_ms_songs_seq, album_name_seq, artist_pop_seq,
           artists_followers_seq, track_pop_seq, artist_genres_seq, params):
    seq_by_name = {
        'artist_name': artist_name_seq, 'track_uri': track_uri_seq,
        'track_name': track_name_seq,
        'duration_ms_songs': duration_ms_songs_seq,
        'album_name': album_name_seq, 'artist_pop': artist_pop_seq,
        'artists_followers': artists_followers_seq,
        'track_pop': track_pop_seq, 'artist_genres': artist_genres_seq,
    }
    scal_idx = [pl_collaborative_idx, pl_pid_idx, duration_ms_seed_idx,
                n_songs_idx, n_artists_idx, n_albums_idx]

    big_names = ['pl_name'] + GATH_SEQ
    big_idx = [pl_name_tokens] + [seq_by_name[n] for n in GATH_SEQ]
    idxs = [jnp.reshape(jnp.swapaxes(a, 0, 1).astype(jnp.int32), (NW, RPW))
            for a in big_idx]
    idx_all = jnp.stack(idxs, axis=1)
    sidx = jnp.concatenate([jnp.reshape(a.astype(jnp.int32), (B // SCH, SCH))
                            for a in scal_idx], axis=0)
    tabs = [params['tab_' + n] for n in big_names]
    stabs = [params['tab_' + n] for n in SCALAR_FEATS]

    oh_idx = [seq_by_name[n].astype(jnp.int32) for n in ONEHOT_SEQ]
    h4 = _tc1(oh_idx,
              [params['tab_' + n] for n in ONEHOT_SEQ],
              [params[n + '_Wx'] for n in ONEHOT_SEQ],
              [params[n + '_Wh'] for n in ONEHOT_SEQ],
              [params[n + '_b'] for n in ONEHOT_SEQ])

    g = _sc_gather(tabs, idx_all, stabs, sidx)
    big_g = [jnp.reshape(a, (L, B, D)) for a in g[:NBIG]]
    scal_g = [jnp.reshape(a, (B, D)) for a in g[NBIG:]]

    dense = [params['dense_W0'], params['dense_b0'],
             params['dense_W1'], params['dense_b1'],
             params['dense_W2'], params['dense_b2']]

    return _tc2(big_g[0], scal_g, big_g[1:], h4,
                [params[n + '_Wx'] for n in GATH_SEQ],
                [params[n + '_Wh'] for n in GATH_SEQ],
                [params[n + '_b'] for n in GATH_SEQ],
                dense)


# speedup vs baseline: 4.0645x; 1.1380x over previous
import functools

import jax
import jax.numpy as jnp
from jax import lax
from jax.experimental import pallas as pl
from jax.experimental.pallas import tpu as pltpu
from jax.experimental.pallas import tpu_sc as plsc

B = 1024
L = 20
D = 128
LAYER_SIZES = [512, 256, 128]
SCALAR_FEATS = ['pl_collaborative', 'pl_pid', 'duration_ms_seed', 'n_songs',
                'n_artists', 'n_albums']
SEQ_FEATS = ['artist_name', 'track_uri', 'track_name', 'duration_ms_songs',
             'album_name', 'artist_pop', 'artists_followers', 'track_pop',
             'artist_genres']
GATH_SEQ = ['artist_name', 'track_uri', 'track_name', 'album_name',
            'artist_genres']
ONEHOT_SEQ = ['duration_ms_songs', 'artist_pop', 'artists_followers',
              'track_pop']
ONEHOT_V = {'duration_ms_songs': 101, 'artist_pop': 11,
            'artists_followers': 11, 'track_pop': 11}
SLOT = {'pl_name': 0, 'pl_collaborative': 1, 'pl_pid': 2,
        'duration_ms_seed': 3, 'n_songs': 4, 'n_artists': 5, 'n_albums': 6,
        'artist_name': 7, 'track_uri': 8, 'track_name': 9,
        'duration_ms_songs': 10, 'album_name': 11, 'artist_pop': 12,
        'artists_followers': 13, 'track_pop': 14, 'artist_genres': 15}

NC = 2
NS = 16
NW = NC * NS
NBIG = 1 + len(GATH_SEQ)
ROWS_BIG = L * B
RPW = ROWS_BIG // NW
SCH = 256
NSC = 6
NSW = NSC * (B // SCH)
NG = len(GATH_SEQ)
NO = len(ONEHOT_SEQ)


def _sc_gather_body(*refs):
    tabs = refs[0:NBIG]
    idx_all = refs[NBIG]
    stabs = refs[NBIG + 1:NBIG + 1 + NSC]
    sidx = refs[NBIG + 1 + NSC]
    outs = refs[NBIG + 2 + NSC:2 * NBIG + 2 + NSC]
    souts = refs[2 * NBIG + 2 + NSC:2 * NBIG + 2 + 2 * NSC]
    rest = refs[2 * NBIG + 2 + 2 * NSC:]
    idx_fs = rest[0:NBIG]
    rows_v, idx_s, gsem, ssem, s2sem = rest[NBIG:]

    w = lax.axis_index("s") * NC + lax.axis_index("c")

    for f in range(NBIG):
        pltpu.sync_copy(idx_all.at[w, f], idx_fs[f])

    for f in range(NBIG):
        g = pltpu.make_async_copy(tabs[f].at[idx_fs[f]], rows_v, gsem)
        g.start()
        g.wait()
        s = pltpu.make_async_copy(rows_v, outs[f].at[w], ssem)
        s.start()
        s.wait()

    rows_sv = rows_v.at[pl.ds(0, SCH)]
    for f in range(NSC):
        for q in range(B // SCH):
            @pl.when(w == f * (B // SCH) + q)
            def _(f=f, q=q):
                pltpu.sync_copy(sidx.at[f * (B // SCH) + q], idx_s)
                g = pltpu.make_async_copy(stabs[f].at[idx_s], rows_sv, s2sem)
                g.start()
                g.wait()
                pltpu.sync_copy(rows_sv, souts[f].at[q])


def _sc_gather(tabs, idx_all, stabs, sidx):
    out_type = ([jax.ShapeDtypeStruct((NW, RPW, D), jnp.float32)
                 for _ in range(NBIG)]
                + [jax.ShapeDtypeStruct((B // SCH, SCH, D), jnp.float32)
                   for _ in range(NSC)])
    mesh = plsc.VectorSubcoreMesh(core_axis_name="c", subcore_axis_name="s")
    fn = pl.kernel(
        _sc_gather_body,
        out_type=out_type,
        mesh=mesh,
        scratch_types=(
            [pltpu.VMEM((RPW,), jnp.int32) for _ in range(NBIG)]
            + [
                pltpu.VMEM((RPW, D), jnp.float32),
                pltpu.VMEM((SCH,), jnp.int32),
                pltpu.SemaphoreType.DMA,
                pltpu.SemaphoreType.DMA,
                pltpu.SemaphoreType.DMA,
            ]
        ),
    )
    return fn(*tabs, idx_all, *stabs, sidx)


def _gru_loop(xz_ref, whg_b, whh_b, n, BB):
    f32 = jnp.float32
    bf16 = jnp.bfloat16

    def step(t, hs):
        new = []
        for f in range(n):
            h = hs[f]
            xt = xz_ref[f, t]
            hg = jnp.dot(h.astype(bf16), whg_b[f], preferred_element_type=f32)
            z = jax.nn.sigmoid(xt[:, :D] + hg[:, :D])
            r = jax.nn.sigmoid(xt[:, D:2 * D] + hg[:, D:])
            hh = jnp.tanh(xt[:, 2 * D:]
                          + jnp.dot((r * h).astype(bf16), whh_b[f],
                                    preferred_element_type=f32))
            new.append(z * h + (1.0 - z) * hh)
        return tuple(new)

    return lax.fori_loop(0, L, step,
                         tuple(jnp.zeros((BB, D), f32) for _ in range(n)))


def _tc1_body(BB, *refs):
    idxs = refs[0:NO]
    tabs = refs[NO:2 * NO]
    wxs = refs[2 * NO:3 * NO]
    whs = refs[3 * NO:4 * NO]
    bs = refs[4 * NO:5 * NO]
    out_ref = refs[5 * NO]

    f32 = jnp.float32
    bf16 = jnp.bfloat16
    whh_b = [whs[f][:, 2 * D:].astype(bf16) for f in range(NO)]
    wc_b = []
    for f in range(NO):
        tabwx = jnp.dot(tabs[f][...], wxs[f][...], preferred_element_type=f32)
        whg_pad = jnp.concatenate(
            [whs[f][:, :2 * D], jnp.zeros((D, D), f32)], axis=1)
        wc_b.append(jnp.concatenate([tabwx, whg_pad], axis=0).astype(bf16))
    ivs = [idxs[f][...] for f in range(NO)]

    hs = [jnp.zeros((BB, D), f32) for _ in range(NO)]
    for t in range(L):
        new = []
        for f, name in enumerate(ONEHOT_SEQ):
            V = ONEHOT_V[name]
            h = hs[f]
            oh = (lax.broadcasted_iota(jnp.int32, (BB, V), 1)
                  == ivs[f][:, t:t + 1]).astype(bf16)
            xin = jnp.concatenate([oh, h.astype(bf16)], axis=1)
            comb = (jnp.dot(xin, wc_b[f], preferred_element_type=f32)
                    + bs[f][...])
            z = jax.nn.sigmoid(comb[:, :D])
            r = jax.nn.sigmoid(comb[:, D:2 * D])
            hh = jnp.tanh(comb[:, 2 * D:]
                          + jnp.dot((r * h).astype(bf16), whh_b[f],
                                    preferred_element_type=f32))
            new.append(z * h + (1.0 - z) * hh)
        hs = new

    for f in range(NO):
        out_ref[:, f * D:(f + 1) * D] = hs[f]


def _tc1(idxs, tabs, wxs, whs, bs):
    BB = 256
    grid = (B // BB,)

    def full(shape):
        n = len(shape)
        return pl.BlockSpec(shape, lambda i, n=n: (0,) * n)

    in_specs = ([pl.BlockSpec((BB, L), lambda i: (i, 0))
                 for _ in range(NO)]
                + [full(t.shape) for t in tabs]
                + [full((D, 3 * D))] * NO
                + [full((D, 3 * D))] * NO
                + [full((3 * D,))] * NO)
    return pl.pallas_call(
        functools.partial(_tc1_body, BB),
        grid=grid,
        in_specs=in_specs,
        out_specs=pl.BlockSpec((BB, NO * D), lambda i: (i, 0)),
        out_shape=jax.ShapeDtypeStruct((B, NO * D), jnp.float32),
    )(*idxs, *tabs, *wxs, *whs, *bs)


def _tc2_body(BB, *refs):
    name_ref = refs[0]
    scal = refs[1:1 + NSC]
    seqs = refs[1 + NSC:1 + NSC + NG]
    h4_ref = refs[1 + NSC + NG]
    wxs = refs[2 + NSC + NG:2 + NSC + 2 * NG]
    whs = refs[2 + NSC + 2 * NG:2 + NSC + 3 * NG]
    bs = refs[2 + NSC + 3 * NG:2 + NSC + 4 * NG]
    W0, b0, W1, b1, W2, b2 = refs[2 + NSC + 4 * NG:8 + NSC + 4 * NG]
    out_ref = refs[8 + NSC + 4 * NG]
    (x_ref,) = refs[9 + NSC + 4 * NG:]

    f32 = jnp.float32
    bf16 = jnp.bfloat16

    x_ref[:, 0:D] = jnp.mean(name_ref[...], axis=0)
    for j in range(NSC):
        x_ref[:, (1 + j) * D:(2 + j) * D] = scal[j][...]
    h4 = h4_ref[...]
    x_ref[:, 10 * D:11 * D] = h4[:, 0:D]
    x_ref[:, 12 * D:15 * D] = h4[:, D:4 * D]

    whh_b = [whs[f][:, 2 * D:].astype(bf16) for f in range(NG)]
    wc_b = []
    for f in range(NG):
        whg_pad = jnp.concatenate(
            [whs[f][:, :2 * D], jnp.zeros((D, D), f32)], axis=1)
        wc_b.append(jnp.concatenate([wxs[f][...], whg_pad],
                                    axis=0).astype(bf16))

    hs = [jnp.zeros((BB, D), f32) for _ in range(NG)]
    for t in range(L):
        new = []
        for f in range(NG):
            h = hs[f]
            xin = jnp.concatenate([seqs[f][t], h], axis=1).astype(bf16)
            comb = (jnp.dot(xin, wc_b[f], preferred_element_type=f32)
                    + bs[f][...])
            z = jax.nn.sigmoid(comb[:, :D])
            r = jax.nn.sigmoid(comb[:, D:2 * D])
            hh = jnp.tanh(comb[:, 2 * D:]
                          + jnp.dot((r * h).astype(bf16), whh_b[f],
                                    preferred_element_type=f32))
            new.append(z * h + (1.0 - z) * hh)
        hs = new
    for f, name in enumerate(GATH_SEQ):
        s = SLOT[name]
        x_ref[:, s * D:(s + 1) * D] = hs[f]

    x = x_ref[...].astype(bf16)
    y = jax.nn.relu(jnp.dot(x, W0[...].astype(bf16),
                            preferred_element_type=f32) + b0[...])
    y = jax.nn.relu(jnp.dot(y.astype(bf16), W1[...].astype(bf16),
                            preferred_element_type=f32) + b1[...])
    out_ref[...] = jnp.dot(y.astype(bf16), W2[...].astype(bf16),
                           preferred_element_type=f32) + b2[...]


def _tc2(name_g, scal_g, seq_g, h4, wxs, whs, bs, dense):
    BB = 256
    grid = (B // BB,)
    time_spec = pl.BlockSpec((L, BB, D), lambda i: (0, i, 0))
    row_spec = pl.BlockSpec((BB, D), lambda i: (i, 0))

    def full(shape):
        n = len(shape)
        return pl.BlockSpec(shape, lambda i, n=n: (0,) * n)

    in_specs = ([time_spec] + [row_spec] * NSC + [time_spec] * NG
                + [pl.BlockSpec((BB, NO * D), lambda i: (i, 0))]
                + [full((D, 3 * D))] * NG + [full((D, 3 * D))] * NG
                + [full((3 * D,))] * NG
                + [full(d.shape) for d in dense])
    return pl.pallas_call(
        functools.partial(_tc2_body, BB),
        grid=grid,
        in_specs=in_specs,
        out_specs=pl.BlockSpec((BB, LAYER_SIZES[-1]), lambda i: (i, 0)),
        out_shape=jax.ShapeDtypeStruct((B, LAYER_SIZES[-1]), jnp.float32),
        scratch_shapes=[
            pltpu.VMEM((BB, 16 * D), jnp.float32),
        ],
    )(name_g, *scal_g, *seq_g, h4, *wxs, *whs, *bs, *dense)


def kernel(pl_name_tokens, pl_collaborative_idx, pl_pid_idx,
           duration_ms_seed_idx, n_songs_idx, n_artists_idx, n_albums_idx,
           artist_name_seq, track_uri_seq, track_name_seq,
           duration_ms_songs_seq, album_name_seq, artist_pop_seq,
           artists_followers_seq, track_pop_seq, artist_genres_seq, params):
    seq_by_name = {
        'artist_name': artist_name_seq, 'track_uri': track_uri_seq,
        'track_name': track_name_seq,
        'duration_ms_songs': duration_ms_songs_seq,
        'album_name': album_name_seq, 'artist_pop': artist_pop_seq,
        'artists_followers': artists_followers_seq,
        'track_pop': track_pop_seq, 'artist_genres': artist_genres_seq,
    }
    scal_idx = [pl_collaborative_idx, pl_pid_idx, duration_ms_seed_idx,
                n_songs_idx, n_artists_idx, n_albums_idx]

    big_names = ['pl_name'] + GATH_SEQ
    big_idx = [pl_name_tokens] + [seq_by_name[n] for n in GATH_SEQ]
    idxs = [jnp.reshape(jnp.swapaxes(a, 0, 1).astype(jnp.int32), (NW, RPW))
            for a in big_idx]
    idx_all = jnp.stack(idxs, axis=1)
    sidx = jnp.concatenate([jnp.reshape(a.astype(jnp.int32), (B // SCH, SCH))
                            for a in scal_idx], axis=0)
    tabs = [params['tab_' + n] for n in big_names]
    stabs = [params['tab_' + n] for n in SCALAR_FEATS]

    oh_idx = [seq_by_name[n].astype(jnp.int32) for n in ONEHOT_SEQ]
    h4 = _tc1(oh_idx,
              [params['tab_' + n] for n in ONEHOT_SEQ],
              [params[n + '_Wx'] for n in ONEHOT_SEQ],
              [params[n + '_Wh'] for n in ONEHOT_SEQ],
              [params[n + '_b'] for n in ONEHOT_SEQ])

    g = _sc_gather(tabs, idx_all, stabs, sidx)
    big_g = [jnp.reshape(a, (L, B, D)) for a in g[:NBIG]]
    scal_g = [jnp.reshape(a, (B, D)) for a in g[NBIG:]]

    dense = [params['dense_W0'], params['dense_b0'],
             params['dense_W1'], params['dense_b1'],
             params['dense_W2'], params['dense_b2']]

    return _tc2(big_g[0], scal_g, big_g[1:], h4,
                [params[n + '_Wx'] for n in GATH_SEQ],
                [params[n + '_Wh'] for n in GATH_SEQ],
                [params[n + '_b'] for n in GATH_SEQ],
                dense)
